# Initial kernel scaffold; baseline (speedup 1.0000x reference)
import jax
import jax.numpy as jnp
from jax.experimental import pallas as pl


def kernel(x, edge_index, batch, W_enc, b_enc, bn_gamma, bn_beta, W1, b1, W2, b2, Wc1, bc1, Wc2, bc2):
    raise NotImplementedError("write your pallas kernel here")


# speedup vs baseline: 10.4859x; 10.4859x over previous
import functools

import jax
import jax.numpy as jnp
from jax import lax
from jax.experimental import pallas as pl
from jax.experimental.pallas import tpu as pltpu
from jax.experimental.pallas import tpu_sc as plsc

N = 50000
E = 800000
IN_DIM = 128
HID = 64
OUT_DIM = 2
G = 64
EPS = 1e-5

NC = 2
NT = 16
NPARTS = 4
HP = HID // NPARTS
PASSES = NPARTS // NC
ROWS_MAIN = 3128
ROWS_TAIL = N - (NT - 1) * ROWS_MAIN
ROWS_EXTRA = ROWS_MAIN - ROWS_TAIL

KE = 2000
EDGES_PER_TILE = E // NT
CONV_CHUNKS = EDGES_PER_TILE // KE

KC = 1000
EDGES_PER_WORKER = E // (NC * NT)
CNT_CHUNKS = EDGES_PER_WORKER // KC
CW = 16

_MESH = dict(core_axis_name="c", subcore_axis_name="s")


def _zero_vmem_rows(ref, nrows, width):
  zv = jnp.zeros((16,), jnp.float32)

  def body(i, _):
    for off in range(0, width, 16):
      ref[i, pl.ds(off, 16)] = zv
    return 0

  lax.fori_loop(0, nrows, body, 0)


def _fill_vmem_rows(ref, nrows, width, value):
  vv = jnp.full((16,), value, jnp.float32)

  def body(i, _):
    for off in range(0, width, 16):
      ref[i, pl.ds(off, 16)] = vv
    return 0

  lax.fori_loop(0, nrows, body, 0)


def _zone_rows(sid):
  return sid * ROWS_MAIN


def _count_body(dst_ref, out_ref, idx_v, ones_v, zrows_v, accum):
  cid = lax.axis_index("c")
  sid = lax.axis_index("s")
  _zero_vmem_rows(zrows_v, ROWS_MAIN, CW)
  _fill_vmem_rows(ones_v, KC, CW, 1.0)
  row0 = _zone_rows(sid)

  @pl.when(sid < NT - 1)
  def _():
    pltpu.sync_copy(zrows_v, accum.at[pl.ds(row0, ROWS_MAIN)])

  @pl.when(sid == NT - 1)
  def _():
    pltpu.sync_copy(
        zrows_v.at[pl.ds(0, ROWS_TAIL)], accum.at[pl.ds(row0, ROWS_TAIL)]
    )

  plsc.subcore_barrier()

  wid = cid * NT + sid

  def step(j, _):
    base = wid * EDGES_PER_WORKER + j * KC
    pltpu.sync_copy(dst_ref.at[pl.ds(base, KC)], idx_v)
    pltpu.sync_copy(ones_v, accum.at[idx_v], add=True)
    return 0

  lax.fori_loop(0, CNT_CHUNKS, step, 0)
  plsc.subcore_barrier()

  @pl.when(sid < NT - 1)
  def _():
    pltpu.sync_copy(
        accum.at[pl.ds(row0, ROWS_MAIN)],
        out_ref.at[cid].at[pl.ds(row0, ROWS_MAIN)],
    )

  @pl.when(sid == NT - 1)
  def _():
    pltpu.sync_copy(
        accum.at[pl.ds(row0, ROWS_TAIL)],
        out_ref.at[cid].at[pl.ds(row0, ROWS_TAIL)],
    )


def _count_call(dst):
  kern = pl.kernel(
      _count_body,
      out_type=jax.ShapeDtypeStruct((NC, N, CW), jnp.float32),
      mesh=plsc.VectorSubcoreMesh(**_MESH),
      compiler_params=pltpu.CompilerParams(use_tc_tiling_on_sc=False),
      scratch_types=[
          pltpu.VMEM((KC,), jnp.int32),
          pltpu.VMEM((KC, CW), jnp.float32),
          pltpu.VMEM((ROWS_MAIN, CW), jnp.float32),
          pltpu.VMEM_SHARED((N, CW), jnp.float32),
      ],
  )
  return kern(dst)


def _conv_body(table_ref, src_ref, dst_ref, out_ref, src_v, dst_v, rows_v,
               accum):
  cid = lax.axis_index("c")
  sid = lax.axis_index("s")
  _zero_vmem_rows(rows_v, KE, HP)
  row0 = _zone_rows(sid)

  for p in range(PASSES):
    part = cid * PASSES + p
    pltpu.sync_copy(rows_v, accum.at[pl.ds(row0, KE)])

    @pl.when(sid < NT - 1)
    def _():
      pltpu.sync_copy(
          rows_v.at[pl.ds(0, ROWS_MAIN - KE)],
          accum.at[pl.ds(row0 + KE, ROWS_MAIN - KE)],
      )

    @pl.when(sid == NT - 1)
    def _():
      pltpu.sync_copy(
          rows_v.at[pl.ds(0, ROWS_TAIL - KE)],
          accum.at[pl.ds(row0 + KE, ROWS_TAIL - KE)],
      )

    plsc.subcore_barrier()

    def step(j, _):
      base = sid * EDGES_PER_TILE + j * KE
      pltpu.sync_copy(src_ref.at[pl.ds(base, KE)], src_v)
      pltpu.sync_copy(dst_ref.at[pl.ds(base, KE)], dst_v)
      pltpu.sync_copy(table_ref.at[part].at[src_v], rows_v)
      pltpu.sync_copy(rows_v, accum.at[dst_v], add=True)
      return 0

    lax.fori_loop(0, CONV_CHUNKS, step, 0)
    plsc.subcore_barrier()

    @pl.when(sid < NT - 1)
    def _():
      pltpu.sync_copy(
          accum.at[pl.ds(row0, ROWS_MAIN)],
          out_ref.at[part].at[pl.ds(row0, ROWS_MAIN)],
      )

    @pl.when(sid == NT - 1)
    def _():
      pltpu.sync_copy(
          accum.at[pl.ds(row0, ROWS_TAIL)],
          out_ref.at[part].at[pl.ds(row0, ROWS_TAIL)],
      )

    if p != PASSES - 1:
      plsc.subcore_barrier()
      _zero_vmem_rows(rows_v, KE, HP)


def _conv_call(table_parts, src, dst):
  kern = pl.kernel(
      _conv_body,
      out_type=jax.ShapeDtypeStruct((NPARTS, N, HP), jnp.float32),
      mesh=plsc.VectorSubcoreMesh(**_MESH),
      compiler_params=pltpu.CompilerParams(use_tc_tiling_on_sc=False),
      scratch_types=[
          pltpu.VMEM((KE,), jnp.int32),
          pltpu.VMEM((KE,), jnp.int32),
          pltpu.VMEM((KE, HP), jnp.float32),
          pltpu.VMEM_SHARED((N, HP), jnp.float32),
      ],
  )
  return kern(table_parts, src, dst)


BN_ROWS = 2000
NBLK = N // BN_ROWS


def _encoder_kernel(x_ref, wenc_ref, benc_ref, a1_ref, c1_ref, out_ref):
  r = jnp.maximum(
      jnp.dot(x_ref[...], wenc_ref[...], preferred_element_type=jnp.float32)
      + benc_ref[...],
      0.0,
  )
  t = jnp.dot(r, a1_ref[...], preferred_element_type=jnp.float32) + c1_ref[...]
  for q in range(NPARTS):
    out_ref[q, :, :] = t[:, q * HP:(q + 1) * HP]


def _encoder_call(x, wencT, benc, A1, c1):
  return pl.pallas_call(
      _encoder_kernel,
      grid=(NBLK,),
      in_specs=[
          pl.BlockSpec((BN_ROWS, IN_DIM), lambda i: (i, 0)),
          pl.BlockSpec((IN_DIM, HID), lambda i: (0, 0)),
          pl.BlockSpec((1, HID), lambda i: (0, 0)),
          pl.BlockSpec((HID, HID), lambda i: (0, 0)),
          pl.BlockSpec((1, HID), lambda i: (0, 0)),
      ],
      out_specs=pl.BlockSpec((NPARTS, BN_ROWS, HP), lambda i: (0, i, 0)),
      out_shape=jax.ShapeDtypeStruct((NPARTS, N, HP), jnp.float32),
  )(x, wencT, benc, A1, c1)


def _meanstats_kernel(s_ref, t_ref, cnt_ref, a_ref, stats_ref):
  i = pl.program_id(0)
  s = jnp.concatenate([s_ref[q] for q in range(NPARTS)], axis=1)
  t = jnp.concatenate([t_ref[q] for q in range(NPARTS)], axis=1)
  cnt = cnt_ref[0, :, 0:1] + cnt_ref[1, :, 0:1] + 1.0
  m = (s + t) / cnt
  a_ref[...] = m
  part = jnp.concatenate(
      [
          jnp.sum(m, axis=0, keepdims=True),
          jnp.sum(m * m, axis=0, keepdims=True),
      ],
      axis=0,
  )

  @pl.when(i == 0)
  def _():
    stats_ref[...] = part

  @pl.when(i > 0)
  def _():
    stats_ref[...] += part


def _meanstats_call(s_parts, t_parts, cnt_parts):
  return pl.pallas_call(
      _meanstats_kernel,
      grid=(NBLK,),
      in_specs=[
          pl.BlockSpec((NPARTS, BN_ROWS, HP), lambda i: (0, i, 0)),
          pl.BlockSpec((NPARTS, BN_ROWS, HP), lambda i: (0, i, 0)),
          pl.BlockSpec((NC, BN_ROWS, CW), lambda i: (0, i, 0)),
      ],
      out_specs=[
          pl.BlockSpec((BN_ROWS, HID), lambda i: (i, 0)),
          pl.BlockSpec((2, HID), lambda i: (0, 0)),
      ],
      out_shape=[
          jax.ShapeDtypeStruct((N, HID), jnp.float32),
          jax.ShapeDtypeStruct((2, HID), jnp.float32),
      ],
  )(s_parts, t_parts, cnt_parts)


def _bnmat_kernel(a_ref, stats_ref, w2_ref, out_ref):
  mean = stats_ref[0:1, :] / N
  var = jnp.maximum(stats_ref[1:2, :] / N - mean * mean, 0.0)
  scale = lax.rsqrt(var + EPS)
  h = jnp.maximum((a_ref[...] - mean) * scale, 0.0)
  t = jnp.dot(h, w2_ref[...], preferred_element_type=jnp.float32)
  for q in range(NPARTS):
    out_ref[q, :, :] = t[:, q * HP:(q + 1) * HP]


def _bnmat_call(a1, stats, W2T):
  return pl.pallas_call(
      _bnmat_kernel,
      grid=(NBLK,),
      in_specs=[
          pl.BlockSpec((BN_ROWS, HID), lambda i: (i, 0)),
          pl.BlockSpec((2, HID), lambda i: (0, 0)),
          pl.BlockSpec((HID, HID), lambda i: (0, 0)),
      ],
      out_specs=pl.BlockSpec((NPARTS, BN_ROWS, HP), lambda i: (0, i, 0)),
      out_shape=jax.ShapeDtypeStruct((NPARTS, N, HP), jnp.float32),
  )(a1, stats, W2T)


def _pool_kernel(s_ref, t_ref, cnt_ref, batch_ref, b2_ref, wc1_ref, bc1_ref,
                 wc2_ref, bc2_ref, out_ref, acc_ref):
  i = pl.program_id(0)
  s = jnp.concatenate([s_ref[q] for q in range(NPARTS)], axis=1)
  t = jnp.concatenate([t_ref[q] for q in range(NPARTS)], axis=1)
  cnt = cnt_ref[0, :, 0:1] + cnt_ref[1, :, 0:1] + 1.0
  h = (s + t) / cnt
  b = jnp.reshape(batch_ref[0], (1, BN_ROWS))
  ohT = (lax.broadcasted_iota(jnp.int32, (G, BN_ROWS), 0) == b).astype(
      jnp.float32
  )
  hcat = jnp.concatenate([h, jnp.ones((BN_ROWS, HID), jnp.float32)], axis=1)
  part = jnp.dot(ohT, hcat, preferred_element_type=jnp.float32)

  @pl.when(i == 0)
  def _():
    acc_ref[...] = part

  @pl.when(i > 0)
  def _():
    acc_ref[...] += part

  @pl.when(i == NBLK - 1)
  def _():
    sums = acc_ref[:, :HID]
    gcnt = acc_ref[:, HID:HID + 1]
    pm = sums / jnp.maximum(gcnt, 1.0)
    pm = pm + jnp.where(gcnt > 0.0, 1.0, 0.0) * b2_ref[...]
    z = jnp.maximum(
        jnp.dot(pm, wc1_ref[...], preferred_element_type=jnp.float32)
        + bc1_ref[...],
        0.0,
    )
    out_ref[...] = (
        jnp.dot(z, wc2_ref[...], preferred_element_type=jnp.float32)
        + bc2_ref[...]
    )


def _pool_call(s_parts, t_parts, cnt_parts, batch, b2, Wc1T, bc1, Wc2T, bc2):
  return pl.pallas_call(
      _pool_kernel,
      grid=(NBLK,),
      in_specs=[
          pl.BlockSpec((NPARTS, BN_ROWS, HP), lambda i: (0, i, 0)),
          pl.BlockSpec((NPARTS, BN_ROWS, HP), lambda i: (0, i, 0)),
          pl.BlockSpec((NC, BN_ROWS, CW), lambda i: (0, i, 0)),
          pl.BlockSpec((1, 1, BN_ROWS), lambda i: (i, 0, 0)),
          pl.BlockSpec((1, HID), lambda i: (0, 0)),
          pl.BlockSpec((HID, HID), lambda i: (0, 0)),
          pl.BlockSpec((1, HID), lambda i: (0, 0)),
          pl.BlockSpec((HID, OUT_DIM), lambda i: (0, 0)),
          pl.BlockSpec((1, OUT_DIM), lambda i: (0, 0)),
      ],
      out_specs=pl.BlockSpec((G, OUT_DIM), lambda i: (0, 0)),
      out_shape=jax.ShapeDtypeStruct((G, OUT_DIM), jnp.float32),
      scratch_shapes=[pltpu.VMEM((G, 2 * HID), jnp.float32)],
  )(s_parts, t_parts, cnt_parts, batch, b2, Wc1T, bc1, Wc2T, bc2)


def kernel(x, edge_index, batch, W_enc, b_enc, bn_gamma, bn_beta,
           W1, b1, W2, b2, Wc1, bc1, Wc2, bc2):
  g = bn_gamma / jnp.sqrt(1.0 + EPS)
  A1 = g[:, None] * W1.T
  c1 = (bn_beta @ W1.T)[None, :]
  src = edge_index[0]
  dst = edge_index[1]
  t1p = _encoder_call(x, W_enc.T, b_enc[None, :], A1, c1)
  cntp = _count_call(dst)
  s1p = _conv_call(t1p, src, dst)
  a1, stats = _meanstats_call(s1p, t1p, cntp)
  t2p = _bnmat_call(a1, stats, W2.T)
  s2p = _conv_call(t2p, src, dst)
  out = _pool_call(s2p, t2p, cntp, batch.reshape(NBLK, 1, BN_ROWS), b2[None, :],
                   Wc1.T, bc1[None, :],
                   Wc2.T, bc2[None, :])
  return out


# speedup vs baseline: 13.1568x; 1.2547x over previous
import jax
import jax.numpy as jnp
from jax import lax
from jax.experimental import pallas as pl
from jax.experimental.pallas import tpu as pltpu
from jax.experimental.pallas import tpu_sc as plsc

N = 50000
E = 800000
IN_DIM = 128
HID = 64
OUT_DIM = 2
G = 64
EPS = 1e-5

NC = 2
NT = 16
NPARTS = 4
HP = HID // NPARTS
PASSES = NPARTS // NC
NP = N // 2
PW = 2 * HID

ROWS_MAIN = 3128
ROWS_TAIL = N - (NT - 1) * ROWS_MAIN

KE = 2000
EDGES_PER_TILE = E // NT
CONV_CHUNKS = EDGES_PER_TILE // KE

KC = 1000
EDGES_PER_WORKER = E // (NC * NT)
CNT_CHUNKS = EDGES_PER_WORKER // KC
CW = 16

PR_MAIN = ROWS_MAIN // 2
PR_TAIL = ROWS_TAIL // 2
PRB = 391
PR_TAIL_REM = PR_TAIL - 3 * PRB

_MESH = dict(core_axis_name="c", subcore_axis_name="s")


def _zero_vmem_rows(ref, nrows, width):
  zv = jnp.zeros((16,), jnp.float32)

  def body(i, _):
    for off in range(0, width, 16):
      ref[i, pl.ds(off, 16)] = zv
    return 0

  lax.fori_loop(0, nrows, body, 0)


def _fill_vmem_rows(ref, nrows, width, value):
  vv = jnp.full((16,), value, jnp.float32)

  def body(i, _):
    for off in range(0, width, 16):
      ref[i, pl.ds(off, 16)] = vv
    return 0

  lax.fori_loop(0, nrows, body, 0)


def _countacc_body(dst_ref, out_ref, idx_v, ones_v, zrows_v, accum):
  cid = lax.axis_index("c")
  sid = lax.axis_index("s")
  _zero_vmem_rows(zrows_v, ROWS_MAIN, CW)
  _fill_vmem_rows(ones_v, KC, CW, 1.0)
  row0 = sid * ROWS_MAIN

  @pl.when(sid < NT - 1)
  def _():
    pltpu.sync_copy(zrows_v, accum.at[pl.ds(row0, ROWS_MAIN)])

  @pl.when(sid == NT - 1)
  def _():
    pltpu.sync_copy(
        zrows_v.at[pl.ds(0, ROWS_TAIL)], accum.at[pl.ds(row0, ROWS_TAIL)]
    )

  plsc.subcore_barrier()

  wid = cid * NT + sid

  def step(j, _):
    base = wid * EDGES_PER_WORKER + j * KC
    pltpu.sync_copy(dst_ref.at[pl.ds(base, KC)], idx_v)
    pltpu.sync_copy(ones_v, accum.at[idx_v], add=True)
    return 0

  lax.fori_loop(0, CNT_CHUNKS, step, 0)
  plsc.subcore_barrier()

  @pl.when(sid < NT - 1)
  def _():
    pltpu.sync_copy(
        accum.at[pl.ds(row0, ROWS_MAIN)],
        out_ref.at[cid].at[pl.ds(row0, ROWS_MAIN)],
    )

  @pl.when(sid == NT - 1)
  def _():
    pltpu.sync_copy(
        accum.at[pl.ds(row0, ROWS_TAIL)],
        out_ref.at[cid].at[pl.ds(row0, ROWS_TAIL)],
    )


def _countacc_call(dst):
  kern = pl.kernel(
      _countacc_body,
      out_type=jax.ShapeDtypeStruct((NC, N, CW), jnp.float32),
      mesh=plsc.VectorSubcoreMesh(**_MESH),
      compiler_params=pltpu.CompilerParams(use_tc_tiling_on_sc=False),
      scratch_types=[
          pltpu.VMEM((KC,), jnp.int32),
          pltpu.VMEM((KC, CW), jnp.float32),
          pltpu.VMEM((ROWS_MAIN, CW), jnp.float32),
          pltpu.VMEM_SHARED((N, CW), jnp.float32),
      ],
  )
  return kern(dst)


def _cntpair_body(cp_ref, out_ref, z0_v, z1_v, pair_v):
  cid = lax.axis_index("c")
  sid = lax.axis_index("s")
  wid = cid * NT + sid
  zone = 784
  half = 392
  pr0 = wid * zone
  tail_rem = NP - 31 * zone - half

  def emit(local_off, nrows):
    n0 = 2 * (pr0 + local_off)
    pltpu.sync_copy(cp_ref.at[0].at[pl.ds(n0, 2 * nrows)],
                    z0_v.at[pl.ds(0, 2 * nrows)])
    pltpu.sync_copy(cp_ref.at[1].at[pl.ds(n0, 2 * nrows)],
                    z1_v.at[pl.ds(0, 2 * nrows)])

    def fill(i, _):
      v0 = z0_v[2 * i, pl.ds(0, CW)] + z1_v[2 * i, pl.ds(0, CW)] + 1.0
      v1 = (
          z0_v[2 * i + 1, pl.ds(0, CW)] + z1_v[2 * i + 1, pl.ds(0, CW)] + 1.0
      )
      for u in range(4):
        pair_v[i, pl.ds(u * CW, CW)] = v0
      for u in range(4, 8):
        pair_v[i, pl.ds(u * CW, CW)] = v1
      return 0

    lax.fori_loop(0, nrows, fill, 0)
    pltpu.sync_copy(
        pair_v.at[pl.ds(0, nrows)], out_ref.at[pl.ds(pr0 + local_off, nrows)]
    )

  emit(0, half)

  @pl.when(wid < NC * NT - 1)
  def _():
    emit(half, half)

  @pl.when(wid == NC * NT - 1)
  def _():
    emit(half, tail_rem)


def _cntpair_call(cparts):
  kern = pl.kernel(
      _cntpair_body,
      out_type=jax.ShapeDtypeStruct((NP, PW), jnp.float32),
      mesh=plsc.VectorSubcoreMesh(**_MESH),
      compiler_params=pltpu.CompilerParams(use_tc_tiling_on_sc=False),
      scratch_types=[
          pltpu.VMEM((2 * 392, CW), jnp.float32),
          pltpu.VMEM((2 * 392, CW), jnp.float32),
          pltpu.VMEM((392, PW), jnp.float32),
      ],
  )
  return kern(cparts)


RC = 1000


def _reformat_body(table_ref, tpart_ref, buf64_v, part_v):
  cid = lax.axis_index("c")
  sid = lax.axis_index("s")
  wid = cid * NT + sid
  zone = 1568
  row0 = wid * zone
  tail = N - 31 * zone

  def reformat_chunk(local_off, nrows):
    pltpu.sync_copy(
        table_ref.at[pl.ds(row0 + local_off, nrows)],
        buf64_v.at[pl.ds(0, nrows)],
    )

    def fill(i, _):
      for q in range(NPARTS):
        part_v[q * RC + i, pl.ds(0, HP)] = buf64_v[i, pl.ds(q * HP, HP)]
      return 0

    lax.fori_loop(0, nrows, fill, 0)
    for q in range(NPARTS):
      pltpu.sync_copy(
          part_v.at[pl.ds(q * RC, nrows)],
          tpart_ref.at[q].at[pl.ds(row0 + local_off, nrows)],
      )

  reformat_chunk(0, 784)

  @pl.when(wid < NC * NT - 1)
  def _():
    reformat_chunk(784, 784)

  @pl.when(wid == NC * NT - 1)
  def _():
    reformat_chunk(784, tail - 784)


def _reformat_call(table):
  kern = pl.kernel(
      _reformat_body,
      out_type=jax.ShapeDtypeStruct((NPARTS, N, HP), jnp.float32),
      mesh=plsc.VectorSubcoreMesh(**_MESH),
      compiler_params=pltpu.CompilerParams(use_tc_tiling_on_sc=False),
      scratch_types=[
          pltpu.VMEM((784, HID), jnp.float32),
          pltpu.VMEM((NPARTS * RC, HP), jnp.float32),
      ],
  )
  return kern(table)


def _conv_body(tpart_ref, src_ref, dst_ref, out_ref, src_v, dst_v, rows_v,
               accum):
  cid = lax.axis_index("c")
  sid = lax.axis_index("s")
  row0 = sid * ROWS_MAIN

  for p in range(PASSES):
    col0 = (cid * PASSES + p) * HP
    _zero_vmem_rows(rows_v, KE, HP)
    pltpu.sync_copy(rows_v, accum.at[pl.ds(row0, KE)])

    @pl.when(sid < NT - 1)
    def _():
      pltpu.sync_copy(
          rows_v.at[pl.ds(0, ROWS_MAIN - KE)],
          accum.at[pl.ds(row0 + KE, ROWS_MAIN - KE)],
      )

    @pl.when(sid == NT - 1)
    def _():
      pltpu.sync_copy(
          rows_v.at[pl.ds(0, ROWS_TAIL - KE)],
          accum.at[pl.ds(row0 + KE, ROWS_TAIL - KE)],
      )

    plsc.subcore_barrier()

    def step(j, _):
      base = sid * EDGES_PER_TILE + j * KE
      pltpu.sync_copy(src_ref.at[pl.ds(base, KE)], src_v)
      pltpu.sync_copy(dst_ref.at[pl.ds(base, KE)], dst_v)
      pltpu.sync_copy(tpart_ref.at[cid * PASSES + p].at[src_v], rows_v)
      pltpu.sync_copy(rows_v, accum.at[dst_v], add=True)
      return 0

    lax.fori_loop(0, CONV_CHUNKS, step, 0)
    plsc.subcore_barrier()

    @pl.when(sid < NT - 1)
    def _():
      pltpu.sync_copy(
          accum.at[pl.ds(row0, ROWS_MAIN)],
          out_ref.at[pl.ds(row0, ROWS_MAIN), pl.ds(col0, HP)],
      )

    @pl.when(sid == NT - 1)
    def _():
      pltpu.sync_copy(
          accum.at[pl.ds(row0, ROWS_TAIL)],
          out_ref.at[pl.ds(row0, ROWS_TAIL), pl.ds(col0, HP)],
      )

    if p != PASSES - 1:
      plsc.subcore_barrier()


def _conv_call(table, src, dst):
  tpart = _reformat_call(table)
  kern = pl.kernel(
      _conv_body,
      out_type=jax.ShapeDtypeStruct((N, HID), jnp.float32),
      mesh=plsc.VectorSubcoreMesh(**_MESH),
      compiler_params=pltpu.CompilerParams(use_tc_tiling_on_sc=False),
      scratch_types=[
          pltpu.VMEM((KE,), jnp.int32),
          pltpu.VMEM((KE,), jnp.int32),
          pltpu.VMEM((KE, HP), jnp.float32),
          pltpu.VMEM_SHARED((N, CW), jnp.float32),
      ],
  )
  return kern(tpart, src, dst)


BP = 1000
NBLK = NP // BP


def _encoder_kernel(x_ref, wenc_ref, benc_ref, a1_ref, c1_ref, out_ref):
  r = jnp.maximum(
      jnp.dot(x_ref[...], wenc_ref[...], preferred_element_type=jnp.float32)
      + benc_ref[...],
      0.0,
  )
  out_ref[...] = (
      jnp.dot(r, a1_ref[...], preferred_element_type=jnp.float32) + c1_ref[...]
  )


def _encoder_call(x_pair, wenc2, benc2, a1d, c1d):
  return pl.pallas_call(
      _encoder_kernel,
      grid=(NBLK,),
      in_specs=[
          pl.BlockSpec((BP, 2 * IN_DIM), lambda i: (i, 0)),
          pl.BlockSpec((2 * IN_DIM, PW), lambda i: (0, 0)),
          pl.BlockSpec((1, PW), lambda i: (0, 0)),
          pl.BlockSpec((PW, PW), lambda i: (0, 0)),
          pl.BlockSpec((1, PW), lambda i: (0, 0)),
      ],
      out_specs=pl.BlockSpec((BP, PW), lambda i: (i, 0)),
      out_shape=jax.ShapeDtypeStruct((NP, PW), jnp.float32),
  )(x_pair, wenc2, benc2, a1d, c1d)


def _meanstats_kernel(s_ref, t_ref, cnt_ref, a_ref, stats_ref):
  i = pl.program_id(0)
  m = (s_ref[...] + t_ref[...]) / cnt_ref[...]
  a_ref[...] = m
  part = jnp.concatenate(
      [
          jnp.sum(m, axis=0, keepdims=True),
          jnp.sum(m * m, axis=0, keepdims=True),
      ],
      axis=0,
  )

  @pl.when(i == 0)
  def _():
    stats_ref[...] = part

  @pl.when(i > 0)
  def _():
    stats_ref[...] += part


def _meanstats_call(s_pair, t_pair, cnt_pair):
  return pl.pallas_call(
      _meanstats_kernel,
      grid=(NBLK,),
      in_specs=[
          pl.BlockSpec((BP, PW), lambda i: (i, 0)),
          pl.BlockSpec((BP, PW), lambda i: (i, 0)),
          pl.BlockSpec((BP, PW), lambda i: (i, 0)),
      ],
      out_specs=[
          pl.BlockSpec((BP, PW), lambda i: (i, 0)),
          pl.BlockSpec((2, PW), lambda i: (0, 0)),
      ],
      out_shape=[
          jax.ShapeDtypeStruct((NP, PW), jnp.float32),
          jax.ShapeDtypeStruct((2, PW), jnp.float32),
      ],
  )(s_pair, t_pair, cnt_pair)


def _bnmat_kernel(a_ref, stats_ref, w2_ref, out_ref):
  mean = (stats_ref[0:1, 0:HID] + stats_ref[0:1, HID:PW]) / N
  msq = (stats_ref[1:2, 0:HID] + stats_ref[1:2, HID:PW]) / N
  var = jnp.maximum(msq - mean * mean, 0.0)
  scale = lax.rsqrt(var + EPS)
  mean2 = jnp.concatenate([mean, mean], axis=1)
  scale2 = jnp.concatenate([scale, scale], axis=1)
  h = jnp.maximum((a_ref[...] - mean2) * scale2, 0.0)
  out_ref[...] = jnp.dot(h, w2_ref[...], preferred_element_type=jnp.float32)


def _bnmat_call(a_pair, stats, w2d):
  return pl.pallas_call(
      _bnmat_kernel,
      grid=(NBLK,),
      in_specs=[
          pl.BlockSpec((BP, PW), lambda i: (i, 0)),
          pl.BlockSpec((2, PW), lambda i: (0, 0)),
          pl.BlockSpec((PW, PW), lambda i: (0, 0)),
      ],
      out_specs=pl.BlockSpec((BP, PW), lambda i: (i, 0)),
      out_shape=jax.ShapeDtypeStruct((NP, PW), jnp.float32),
  )(a_pair, stats, w2d)


def _pool_kernel(s_ref, t_ref, cnt_ref, be_ref, bo_ref, b2_ref, wc1_ref,
                 bc1_ref, wc2_ref, bc2_ref, out_ref, acc_ref):
  i = pl.program_id(0)
  h = (s_ref[...] + t_ref[...]) / cnt_ref[...]
  be = jnp.reshape(be_ref[0], (1, BP))
  bo = jnp.reshape(bo_ref[0], (1, BP))
  giota = lax.broadcasted_iota(jnp.int32, (G, BP), 0)
  ohe = (giota == be).astype(jnp.float32)
  oho = (giota == bo).astype(jnp.float32)
  ones = jnp.ones((BP, HID), jnp.float32)
  he = jnp.concatenate([h[:, 0:HID], ones], axis=1)
  ho = jnp.concatenate([h[:, HID:PW], ones], axis=1)
  part = (
      jnp.dot(ohe, he, preferred_element_type=jnp.float32)
      + jnp.dot(oho, ho, preferred_element_type=jnp.float32)
  )

  @pl.when(i == 0)
  def _():
    acc_ref[...] = part

  @pl.when(i > 0)
  def _():
    acc_ref[...] += part

  @pl.when(i == NBLK - 1)
  def _():
    sums = acc_ref[:, 0:HID]
    gcnt = acc_ref[:, HID:HID + 1]
    pm = sums / jnp.maximum(gcnt, 1.0)
    pm = pm + jnp.where(gcnt > 0.0, 1.0, 0.0) * b2_ref[...]
    z = jnp.maximum(
        jnp.dot(pm, wc1_ref[...], preferred_element_type=jnp.float32)
        + bc1_ref[...],
        0.0,
    )
    out_ref[...] = (
        jnp.dot(z, wc2_ref[...], preferred_element_type=jnp.float32)
        + bc2_ref[...]
    )


def _pool_call(s_pair, t_pair, cnt_pair, batch_e, batch_o, b2, Wc1T, bc1,
               Wc2T, bc2):
  return pl.pallas_call(
      _pool_kernel,
      grid=(NBLK,),
      in_specs=[
          pl.BlockSpec((BP, PW), lambda i: (i, 0)),
          pl.BlockSpec((BP, PW), lambda i: (i, 0)),
          pl.BlockSpec((BP, PW), lambda i: (i, 0)),
          pl.BlockSpec((1, 1, BP), lambda i: (i, 0, 0)),
          pl.BlockSpec((1, 1, BP), lambda i: (i, 0, 0)),
          pl.BlockSpec((1, HID), lambda i: (0, 0)),
          pl.BlockSpec((HID, HID), lambda i: (0, 0)),
          pl.BlockSpec((1, HID), lambda i: (0, 0)),
          pl.BlockSpec((HID, OUT_DIM), lambda i: (0, 0)),
          pl.BlockSpec((1, OUT_DIM), lambda i: (0, 0)),
      ],
      out_specs=pl.BlockSpec((G, OUT_DIM), lambda i: (0, 0)),
      out_shape=jax.ShapeDtypeStruct((G, OUT_DIM), jnp.float32),
      scratch_shapes=[pltpu.VMEM((G, 2 * HID), jnp.float32)],
  )(s_pair, t_pair, cnt_pair, batch_e, batch_o, b2, Wc1T, bc1, Wc2T, bc2)


def _blockdiag(w):
  z = jnp.zeros_like(w)
  return jnp.concatenate(
      [jnp.concatenate([w, z], axis=1), jnp.concatenate([z, w], axis=1)],
      axis=0,
  )


def kernel(x, edge_index, batch, W_enc, b_enc, bn_gamma, bn_beta,
           W1, b1, W2, b2, Wc1, bc1, Wc2, bc2):
  g = bn_gamma / jnp.sqrt(1.0 + EPS)
  A1 = g[:, None] * W1.T
  c1 = bn_beta @ W1.T
  src = edge_index[0]
  dst = edge_index[1]

  x_pair = x.reshape(NP, 2 * IN_DIM)
  wenc2 = _blockdiag(W_enc.T)
  benc2 = jnp.tile(b_enc, 2)[None, :]
  a1d = _blockdiag(A1)
  c1d = jnp.tile(c1, 2)[None, :]
  w2d = _blockdiag(W2.T)
  batch_e = batch[0::2].reshape(NBLK, 1, BP)
  batch_o = batch[1::2].reshape(NBLK, 1, BP)

  t1_pair = _encoder_call(x_pair, wenc2, benc2, a1d, c1d)
  cnt_pair = _cntpair_call(_countacc_call(dst))
  s1_pair = _conv_call(t1_pair.reshape(N, HID), src, dst).reshape(NP, PW)
  a1_pair, stats = _meanstats_call(s1_pair, t1_pair, cnt_pair)
  t2_pair = _bnmat_call(a1_pair, stats, w2d)
  s2_pair = _conv_call(t2_pair.reshape(N, HID), src, dst).reshape(NP, PW)
  out = _pool_call(s2_pair, t2_pair, cnt_pair, batch_e, batch_o, b2[None, :],
                   Wc1.T, bc1[None, :], Wc2.T, bc2[None, :])
  return out


# speedup vs baseline: 15.8565x; 1.2052x over previous
import jax
import jax.numpy as jnp
from jax import lax
from jax.experimental import pallas as pl
from jax.experimental.pallas import tpu as pltpu
from jax.experimental.pallas import tpu_sc as plsc

N = 50000
E = 800000
IN_DIM = 128
HID = 64
OUT_DIM = 2
G = 64
EPS = 1e-5

NC = 2
NT = 16
NPARTS = 4
HP = HID // NPARTS
PASSES = NPARTS // NC
NP = N // 2
PW = 2 * HID

ROWS_MAIN = 3128
ROWS_TAIL = N - (NT - 1) * ROWS_MAIN

KE = 2000
EDGES_PER_TILE = E // NT
CONV_CHUNKS = EDGES_PER_TILE // KE

KC = 1000
EDGES_PER_WORKER = E // (NC * NT)
CNT_CHUNKS = EDGES_PER_WORKER // KC
CW = 16

PR_MAIN = ROWS_MAIN // 2
PR_TAIL = ROWS_TAIL // 2
PRB = 391
PR_TAIL_REM = PR_TAIL - 3 * PRB

_MESH = dict(core_axis_name="c", subcore_axis_name="s")


def _zero_vmem_rows(ref, nrows, width):
  zv = jnp.zeros((16,), jnp.float32)

  def body(i, _):
    for off in range(0, width, 16):
      ref[i, pl.ds(off, 16)] = zv
    return 0

  lax.fori_loop(0, nrows, body, 0)


def _fill_vmem_rows(ref, nrows, width, value):
  vv = jnp.full((16,), value, jnp.float32)

  def body(i, _):
    for off in range(0, width, 16):
      ref[i, pl.ds(off, 16)] = vv
    return 0

  lax.fori_loop(0, nrows, body, 0)


def _countacc_body(dst_ref, out_ref, idx_v, ones_v, zrows_v, accum):
  cid = lax.axis_index("c")
  sid = lax.axis_index("s")
  _zero_vmem_rows(zrows_v, ROWS_MAIN, CW)
  _fill_vmem_rows(ones_v, KC, CW, 1.0)
  row0 = sid * ROWS_MAIN

  @pl.when(sid < NT - 1)
  def _():
    pltpu.sync_copy(zrows_v, accum.at[pl.ds(row0, ROWS_MAIN)])

  @pl.when(sid == NT - 1)
  def _():
    pltpu.sync_copy(
        zrows_v.at[pl.ds(0, ROWS_TAIL)], accum.at[pl.ds(row0, ROWS_TAIL)]
    )

  plsc.subcore_barrier()

  wid = cid * NT + sid

  def step(j, _):
    base = wid * EDGES_PER_WORKER + j * KC
    pltpu.sync_copy(dst_ref.at[pl.ds(base, KC)], idx_v)
    pltpu.sync_copy(ones_v, accum.at[idx_v], add=True)
    return 0

  lax.fori_loop(0, CNT_CHUNKS, step, 0)
  plsc.subcore_barrier()

  @pl.when(sid < NT - 1)
  def _():
    pltpu.sync_copy(
        accum.at[pl.ds(row0, ROWS_MAIN)],
        out_ref.at[cid].at[pl.ds(row0, ROWS_MAIN)],
    )

  @pl.when(sid == NT - 1)
  def _():
    pltpu.sync_copy(
        accum.at[pl.ds(row0, ROWS_TAIL)],
        out_ref.at[cid].at[pl.ds(row0, ROWS_TAIL)],
    )


def _countacc_call(dst):
  kern = pl.kernel(
      _countacc_body,
      out_type=jax.ShapeDtypeStruct((NC, N, CW), jnp.float32),
      mesh=plsc.VectorSubcoreMesh(**_MESH),
      compiler_params=pltpu.CompilerParams(use_tc_tiling_on_sc=False),
      scratch_types=[
          pltpu.VMEM((KC,), jnp.int32),
          pltpu.VMEM((KC, CW), jnp.float32),
          pltpu.VMEM((ROWS_MAIN, CW), jnp.float32),
          pltpu.VMEM_SHARED((N, CW), jnp.float32),
      ],
  )
  return kern(dst)


def _cntpair_body(cp_ref, out_ref, z0_v, z1_v, pair_v):
  cid = lax.axis_index("c")
  sid = lax.axis_index("s")
  wid = cid * NT + sid
  zone = 784
  half = 392
  pr0 = wid * zone
  tail_rem = NP - 31 * zone - half

  def emit(local_off, nrows):
    n0 = 2 * (pr0 + local_off)
    pltpu.sync_copy(cp_ref.at[0].at[pl.ds(n0, 2 * nrows)],
                    z0_v.at[pl.ds(0, 2 * nrows)])
    pltpu.sync_copy(cp_ref.at[1].at[pl.ds(n0, 2 * nrows)],
                    z1_v.at[pl.ds(0, 2 * nrows)])

    def fill(i, _):
      v0 = z0_v[2 * i, pl.ds(0, CW)] + z1_v[2 * i, pl.ds(0, CW)] + 1.0
      v1 = (
          z0_v[2 * i + 1, pl.ds(0, CW)] + z1_v[2 * i + 1, pl.ds(0, CW)] + 1.0
      )
      for u in range(4):
        pair_v[i, pl.ds(u * CW, CW)] = v0
      for u in range(4, 8):
        pair_v[i, pl.ds(u * CW, CW)] = v1
      return 0

    lax.fori_loop(0, nrows, fill, 0)
    pltpu.sync_copy(
        pair_v.at[pl.ds(0, nrows)], out_ref.at[pl.ds(pr0 + local_off, nrows)]
    )

  emit(0, half)

  @pl.when(wid < NC * NT - 1)
  def _():
    emit(half, half)

  @pl.when(wid == NC * NT - 1)
  def _():
    emit(half, tail_rem)


def _cntpair_call(cparts):
  kern = pl.kernel(
      _cntpair_body,
      out_type=jax.ShapeDtypeStruct((NP, PW), jnp.float32),
      mesh=plsc.VectorSubcoreMesh(**_MESH),
      compiler_params=pltpu.CompilerParams(use_tc_tiling_on_sc=False),
      scratch_types=[
          pltpu.VMEM((2 * 392, CW), jnp.float32),
          pltpu.VMEM((2 * 392, CW), jnp.float32),
          pltpu.VMEM((392, PW), jnp.float32),
      ],
  )
  return kern(cparts)


RC = 1000


def _reformat_body(table_ref, tpart_ref, buf64_v, part_v):
  cid = lax.axis_index("c")
  sid = lax.axis_index("s")
  wid = cid * NT + sid
  zone = 1568
  row0 = wid * zone
  tail = N - 31 * zone

  def reformat_chunk(local_off, nrows):
    pltpu.sync_copy(
        table_ref.at[pl.ds(row0 + local_off, nrows)],
        buf64_v.at[pl.ds(0, nrows)],
    )

    def fill(i, _):
      for q in range(NPARTS):
        part_v[q * RC + i, pl.ds(0, HP)] = buf64_v[i, pl.ds(q * HP, HP)]
      return 0

    lax.fori_loop(0, nrows, fill, 0)
    for q in range(NPARTS):
      pltpu.sync_copy(
          part_v.at[pl.ds(q * RC, nrows)],
          tpart_ref.at[q].at[pl.ds(row0 + local_off, nrows)],
      )

  reformat_chunk(0, 784)

  @pl.when(wid < NC * NT - 1)
  def _():
    reformat_chunk(784, 784)

  @pl.when(wid == NC * NT - 1)
  def _():
    reformat_chunk(784, tail - 784)


def _reformat_call(table):
  kern = pl.kernel(
      _reformat_body,
      out_type=jax.ShapeDtypeStruct((NPARTS, N, HP), jnp.float32),
      mesh=plsc.VectorSubcoreMesh(**_MESH),
      compiler_params=pltpu.CompilerParams(use_tc_tiling_on_sc=False),
      scratch_types=[
          pltpu.VMEM((784, HID), jnp.float32),
          pltpu.VMEM((NPARTS * RC, HP), jnp.float32),
      ],
  )
  return kern(table)


def _conv_body(tpart_ref, src_ref, dst_ref, out_ref, src_v0, src_v1, dst_v0,
               dst_v1, rows_v0, rows_v1, sem_i0, sem_i1, sem_g, sem_s0,
               sem_s1, accum):
  cid = lax.axis_index("c")
  sid = lax.axis_index("s")
  row0 = sid * ROWS_MAIN
  src_v = (src_v0, src_v1)
  dst_v = (dst_v0, dst_v1)
  rows_v = (rows_v0, rows_v1)
  sem_i = (sem_i0, sem_i1)
  sem_s = (sem_s0, sem_s1)

  def idx_start(j, b):
    base = sid * EDGES_PER_TILE + j * KE
    pltpu.make_async_copy(
        src_ref.at[pl.ds(base, KE)], src_v[b], sem_i[b]
    ).start()
    pltpu.make_async_copy(
        dst_ref.at[pl.ds(base, KE)], dst_v[b], sem_i[b]
    ).start()

  def idx_wait(j, b):
    base = sid * EDGES_PER_TILE + j * KE
    pltpu.make_async_copy(
        src_ref.at[pl.ds(base, KE)], src_v[b], sem_i[b]
    ).wait()
    pltpu.make_async_copy(
        dst_ref.at[pl.ds(base, KE)], dst_v[b], sem_i[b]
    ).wait()

  for p in range(PASSES):
    part = cid * PASSES + p
    col0 = part * HP
    _zero_vmem_rows(rows_v0, KE, HP)
    pltpu.sync_copy(rows_v0, accum.at[pl.ds(row0, KE)])

    @pl.when(sid < NT - 1)
    def _():
      pltpu.sync_copy(
          rows_v0.at[pl.ds(0, ROWS_MAIN - KE)],
          accum.at[pl.ds(row0 + KE, ROWS_MAIN - KE)],
      )

    @pl.when(sid == NT - 1)
    def _():
      pltpu.sync_copy(
          rows_v0.at[pl.ds(0, ROWS_TAIL - KE)],
          accum.at[pl.ds(row0 + KE, ROWS_TAIL - KE)],
      )

    plsc.subcore_barrier()

    idx_start(0, 0)

    def step(j, _):
      for b in range(2):

        @pl.when(j % 2 == b)
        def _():
          nb = 1 - b
          idx_wait(j, b)
          pltpu.async_copy(
              tpart_ref.at[part].at[src_v[b]], rows_v[b], sem_g
          ).wait()

          @pl.when(j >= 1)
          def _():
            pltpu.make_async_copy(
                rows_v[nb], accum.at[dst_v[nb]], sem_s[nb]
            ).wait()

          @pl.when(j < CONV_CHUNKS - 1)
          def _():
            idx_start(j + 1, nb)

          pltpu.make_async_copy(
              rows_v[b], accum.at[dst_v[b]], sem_s[b]
          ).start(add=True)

      return 0

    lax.fori_loop(0, CONV_CHUNKS, step, 0)
    lastb = (CONV_CHUNKS - 1) % 2
    pltpu.make_async_copy(
        rows_v[lastb], accum.at[dst_v[lastb]], sem_s[lastb]
    ).wait()
    plsc.subcore_barrier()

    @pl.when(sid < NT - 1)
    def _():
      pltpu.sync_copy(
          accum.at[pl.ds(row0, ROWS_MAIN)],
          out_ref.at[pl.ds(row0, ROWS_MAIN), pl.ds(col0, HP)],
      )

    @pl.when(sid == NT - 1)
    def _():
      pltpu.sync_copy(
          accum.at[pl.ds(row0, ROWS_TAIL)],
          out_ref.at[pl.ds(row0, ROWS_TAIL), pl.ds(col0, HP)],
      )

    if p != PASSES - 1:
      plsc.subcore_barrier()


def _conv_call(table, src, dst):
  tpart = _reformat_call(table)
  kern = pl.kernel(
      _conv_body,
      out_type=jax.ShapeDtypeStruct((N, HID), jnp.float32),
      mesh=plsc.VectorSubcoreMesh(**_MESH),
      compiler_params=pltpu.CompilerParams(use_tc_tiling_on_sc=False),
      scratch_types=[
          pltpu.VMEM((KE,), jnp.int32),
          pltpu.VMEM((KE,), jnp.int32),
          pltpu.VMEM((KE,), jnp.int32),
          pltpu.VMEM((KE,), jnp.int32),
          pltpu.VMEM((KE, HP), jnp.float32),
          pltpu.VMEM((KE, HP), jnp.float32),
          pltpu.SemaphoreType.DMA,
          pltpu.SemaphoreType.DMA,
          pltpu.SemaphoreType.DMA,
          pltpu.SemaphoreType.DMA,
          pltpu.SemaphoreType.DMA,
          pltpu.VMEM_SHARED((N, CW), jnp.float32),
      ],
  )
  return kern(tpart, src, dst)


BP = 1000
NBLK = NP // BP


def _encoder_kernel(x_ref, wenc_ref, benc_ref, a1_ref, c1_ref, out_ref):
  r = jnp.maximum(
      jnp.dot(x_ref[...], wenc_ref[...], preferred_element_type=jnp.float32)
      + benc_ref[...],
      0.0,
  )
  out_ref[...] = (
      jnp.dot(r, a1_ref[...], preferred_element_type=jnp.float32) + c1_ref[...]
  )


def _encoder_call(x_pair, wenc2, benc2, a1d, c1d):
  return pl.pallas_call(
      _encoder_kernel,
      grid=(NBLK,),
      in_specs=[
          pl.BlockSpec((BP, 2 * IN_DIM), lambda i: (i, 0)),
          pl.BlockSpec((2 * IN_DIM, PW), lambda i: (0, 0)),
          pl.BlockSpec((1, PW), lambda i: (0, 0)),
          pl.BlockSpec((PW, PW), lambda i: (0, 0)),
          pl.BlockSpec((1, PW), lambda i: (0, 0)),
      ],
      out_specs=pl.BlockSpec((BP, PW), lambda i: (i, 0)),
      out_shape=jax.ShapeDtypeStruct((NP, PW), jnp.float32),
  )(x_pair, wenc2, benc2, a1d, c1d)


def _meanstats_kernel(s_ref, t_ref, cnt_ref, a_ref, stats_ref):
  i = pl.program_id(0)
  m = (s_ref[...] + t_ref[...]) / cnt_ref[...]
  a_ref[...] = m
  part = jnp.concatenate(
      [
          jnp.sum(m, axis=0, keepdims=True),
          jnp.sum(m * m, axis=0, keepdims=True),
      ],
      axis=0,
  )

  @pl.when(i == 0)
  def _():
    stats_ref[...] = part

  @pl.when(i > 0)
  def _():
    stats_ref[...] += part


def _meanstats_call(s_pair, t_pair, cnt_pair):
  return pl.pallas_call(
      _meanstats_kernel,
      grid=(NBLK,),
      in_specs=[
          pl.BlockSpec((BP, PW), lambda i: (i, 0)),
          pl.BlockSpec((BP, PW), lambda i: (i, 0)),
          pl.BlockSpec((BP, PW), lambda i: (i, 0)),
      ],
      out_specs=[
          pl.BlockSpec((BP, PW), lambda i: (i, 0)),
          pl.BlockSpec((2, PW), lambda i: (0, 0)),
      ],
      out_shape=[
          jax.ShapeDtypeStruct((NP, PW), jnp.float32),
          jax.ShapeDtypeStruct((2, PW), jnp.float32),
      ],
  )(s_pair, t_pair, cnt_pair)


def _bnmat_kernel(a_ref, stats_ref, w2_ref, out_ref):
  mean = (stats_ref[0:1, 0:HID] + stats_ref[0:1, HID:PW]) / N
  msq = (stats_ref[1:2, 0:HID] + stats_ref[1:2, HID:PW]) / N
  var = jnp.maximum(msq - mean * mean, 0.0)
  scale = lax.rsqrt(var + EPS)
  mean2 = jnp.concatenate([mean, mean], axis=1)
  scale2 = jnp.concatenate([scale, scale], axis=1)
  h = jnp.maximum((a_ref[...] - mean2) * scale2, 0.0)
  out_ref[...] = jnp.dot(h, w2_ref[...], preferred_element_type=jnp.float32)


def _bnmat_call(a_pair, stats, w2d):
  return pl.pallas_call(
      _bnmat_kernel,
      grid=(NBLK,),
      in_specs=[
          pl.BlockSpec((BP, PW), lambda i: (i, 0)),
          pl.BlockSpec((2, PW), lambda i: (0, 0)),
          pl.BlockSpec((PW, PW), lambda i: (0, 0)),
      ],
      out_specs=pl.BlockSpec((BP, PW), lambda i: (i, 0)),
      out_shape=jax.ShapeDtypeStruct((NP, PW), jnp.float32),
  )(a_pair, stats, w2d)


def _pool_kernel(s_ref, t_ref, cnt_ref, be_ref, bo_ref, b2_ref, wc1_ref,
                 bc1_ref, wc2_ref, bc2_ref, out_ref, acc_ref):
  i = pl.program_id(0)
  h = (s_ref[...] + t_ref[...]) / cnt_ref[...]
  be = jnp.reshape(be_ref[0], (1, BP))
  bo = jnp.reshape(bo_ref[0], (1, BP))
  giota = lax.broadcasted_iota(jnp.int32, (G, BP), 0)
  ohe = (giota == be).astype(jnp.float32)
  oho = (giota == bo).astype(jnp.float32)
  ones = jnp.ones((BP, HID), jnp.float32)
  he = jnp.concatenate([h[:, 0:HID], ones], axis=1)
  ho = jnp.concatenate([h[:, HID:PW], ones], axis=1)
  part = (
      jnp.dot(ohe, he, preferred_element_type=jnp.float32)
      + jnp.dot(oho, ho, preferred_element_type=jnp.float32)
  )

  @pl.when(i == 0)
  def _():
    acc_ref[...] = part

  @pl.when(i > 0)
  def _():
    acc_ref[...] += part

  @pl.when(i == NBLK - 1)
  def _():
    sums = acc_ref[:, 0:HID]
    gcnt = acc_ref[:, HID:HID + 1]
    pm = sums / jnp.maximum(gcnt, 1.0)
    pm = pm + jnp.where(gcnt > 0.0, 1.0, 0.0) * b2_ref[...]
    z = jnp.maximum(
        jnp.dot(pm, wc1_ref[...], preferred_element_type=jnp.float32)
        + bc1_ref[...],
        0.0,
    )
    out_ref[...] = (
        jnp.dot(z, wc2_ref[...], preferred_element_type=jnp.float32)
        + bc2_ref[...]
    )


def _pool_call(s_pair, t_pair, cnt_pair, batch_e, batch_o, b2, Wc1T, bc1,
               Wc2T, bc2):
  return pl.pallas_call(
      _pool_kernel,
      grid=(NBLK,),
      in_specs=[
          pl.BlockSpec((BP, PW), lambda i: (i, 0)),
          pl.BlockSpec((BP, PW), lambda i: (i, 0)),
          pl.BlockSpec((BP, PW), lambda i: (i, 0)),
          pl.BlockSpec((1, 1, BP), lambda i: (i, 0, 0)),
          pl.BlockSpec((1, 1, BP), lambda i: (i, 0, 0)),
          pl.BlockSpec((1, HID), lambda i: (0, 0)),
          pl.BlockSpec((HID, HID), lambda i: (0, 0)),
          pl.BlockSpec((1, HID), lambda i: (0, 0)),
          pl.BlockSpec((HID, OUT_DIM), lambda i: (0, 0)),
          pl.BlockSpec((1, OUT_DIM), lambda i: (0, 0)),
      ],
      out_specs=pl.BlockSpec((G, OUT_DIM), lambda i: (0, 0)),
      out_shape=jax.ShapeDtypeStruct((G, OUT_DIM), jnp.float32),
      scratch_shapes=[pltpu.VMEM((G, 2 * HID), jnp.float32)],
  )(s_pair, t_pair, cnt_pair, batch_e, batch_o, b2, Wc1T, bc1, Wc2T, bc2)


def _blockdiag(w):
  z = jnp.zeros_like(w)
  return jnp.concatenate(
      [jnp.concatenate([w, z], axis=1), jnp.concatenate([z, w], axis=1)],
      axis=0,
  )


def kernel(x, edge_index, batch, W_enc, b_enc, bn_gamma, bn_beta,
           W1, b1, W2, b2, Wc1, bc1, Wc2, bc2):
  g = bn_gamma / jnp.sqrt(1.0 + EPS)
  A1 = g[:, None] * W1.T
  c1 = bn_beta @ W1.T
  src = edge_index[0]
  dst = edge_index[1]

  x_pair = x.reshape(NP, 2 * IN_DIM)
  wenc2 = _blockdiag(W_enc.T)
  benc2 = jnp.tile(b_enc, 2)[None, :]
  a1d = _blockdiag(A1)
  c1d = jnp.tile(c1, 2)[None, :]
  w2d = _blockdiag(W2.T)
  batch_e = batch[0::2].reshape(NBLK, 1, BP)
  batch_o = batch[1::2].reshape(NBLK, 1, BP)

  t1_pair = _encoder_call(x_pair, wenc2, benc2, a1d, c1d)
  cnt_pair = _cntpair_call(_countacc_call(dst))
  s1_pair = _conv_call(t1_pair.reshape(N, HID), src, dst).reshape(NP, PW)
  a1_pair, stats = _meanstats_call(s1_pair, t1_pair, cnt_pair)
  t2_pair = _bnmat_call(a1_pair, stats, w2d)
  s2_pair = _conv_call(t2_pair.reshape(N, HID), src, dst).reshape(NP, PW)
  out = _pool_call(s2_pair, t2_pair, cnt_pair, batch_e, batch_o, b2[None, :],
                   Wc1.T, bc1[None, :], Wc2.T, bc2[None, :])
  return out


# speedup vs baseline: 16.3310x; 1.0299x over previous
import jax
import jax.numpy as jnp
from jax import lax
from jax.experimental import pallas as pl
from jax.experimental.pallas import tpu as pltpu
from jax.experimental.pallas import tpu_sc as plsc

N = 50000
E = 800000
IN_DIM = 128
HID = 64
OUT_DIM = 2
G = 64
EPS = 1e-5

NC = 2
NT = 16
NPARTS = 4
HP = HID // NPARTS
PASSES = NPARTS // NC
NP = N // 2
PW = 2 * HID

ROWS_MAIN = 3128
ROWS_TAIL = N - (NT - 1) * ROWS_MAIN

KE = 2000
EDGES_PER_TILE = E // NT
CONV_CHUNKS = EDGES_PER_TILE // KE

KC = 1000
EDGES_PER_WORKER = E // (NC * NT)
CNT_CHUNKS = EDGES_PER_WORKER // KC
CW = 16

PR_MAIN = ROWS_MAIN // 2
PR_TAIL = ROWS_TAIL // 2
PRB = 391
PR_TAIL_REM = PR_TAIL - 3 * PRB

_MESH = dict(core_axis_name="c", subcore_axis_name="s")


def _zero_vmem_rows(ref, nrows, width):
  zv = jnp.zeros((16,), jnp.float32)

  def body(i, _):
    for off in range(0, width, 16):
      ref[i, pl.ds(off, 16)] = zv
    return 0

  lax.fori_loop(0, nrows, body, 0)


def _fill_vmem_rows(ref, nrows, width, value):
  vv = jnp.full((16,), value, jnp.float32)

  def body(i, _):
    for off in range(0, width, 16):
      ref[i, pl.ds(off, 16)] = vv
    return 0

  lax.fori_loop(0, nrows, body, 0)


def _countacc_body(dst_ref, out_ref, idx_v, ones_v, zrows_v, accum):
  cid = lax.axis_index("c")
  sid = lax.axis_index("s")
  _zero_vmem_rows(zrows_v, ROWS_MAIN, CW)
  _fill_vmem_rows(ones_v, KC, CW, 1.0)
  row0 = sid * ROWS_MAIN

  @pl.when(sid < NT - 1)
  def _():
    pltpu.sync_copy(zrows_v, accum.at[pl.ds(row0, ROWS_MAIN)])

  @pl.when(sid == NT - 1)
  def _():
    pltpu.sync_copy(
        zrows_v.at[pl.ds(0, ROWS_TAIL)], accum.at[pl.ds(row0, ROWS_TAIL)]
    )

  plsc.subcore_barrier()

  wid = cid * NT + sid

  def step(j, _):
    base = wid * EDGES_PER_WORKER + j * KC
    pltpu.sync_copy(dst_ref.at[pl.ds(base, KC)], idx_v)
    pltpu.sync_copy(ones_v, accum.at[idx_v], add=True)
    return 0

  lax.fori_loop(0, CNT_CHUNKS, step, 0)
  plsc.subcore_barrier()

  @pl.when(sid < NT - 1)
  def _():
    pltpu.sync_copy(
        accum.at[pl.ds(row0, ROWS_MAIN)],
        out_ref.at[cid].at[pl.ds(row0, ROWS_MAIN)],
    )

  @pl.when(sid == NT - 1)
  def _():
    pltpu.sync_copy(
        accum.at[pl.ds(row0, ROWS_TAIL)],
        out_ref.at[cid].at[pl.ds(row0, ROWS_TAIL)],
    )


def _countacc_call(dst):
  kern = pl.kernel(
      _countacc_body,
      out_type=jax.ShapeDtypeStruct((NC, N, CW), jnp.float32),
      mesh=plsc.VectorSubcoreMesh(**_MESH),
      compiler_params=pltpu.CompilerParams(use_tc_tiling_on_sc=False),
      scratch_types=[
          pltpu.VMEM((KC,), jnp.int32),
          pltpu.VMEM((KC, CW), jnp.float32),
          pltpu.VMEM((ROWS_MAIN, CW), jnp.float32),
          pltpu.VMEM_SHARED((N, CW), jnp.float32),
      ],
  )
  return kern(dst)


def _cntpair_body(cp_ref, out_ref, z0_v, z1_v, pair_v):
  cid = lax.axis_index("c")
  sid = lax.axis_index("s")
  wid = cid * NT + sid
  zone = 784
  half = 392
  pr0 = wid * zone
  tail_rem = NP - 31 * zone - half

  def emit(local_off, nrows):
    n0 = 2 * (pr0 + local_off)
    pltpu.sync_copy(cp_ref.at[0].at[pl.ds(n0, 2 * nrows)],
                    z0_v.at[pl.ds(0, 2 * nrows)])
    pltpu.sync_copy(cp_ref.at[1].at[pl.ds(n0, 2 * nrows)],
                    z1_v.at[pl.ds(0, 2 * nrows)])

    def fill(i, _):
      v0 = z0_v[2 * i, pl.ds(0, CW)] + z1_v[2 * i, pl.ds(0, CW)] + 1.0
      v1 = (
          z0_v[2 * i + 1, pl.ds(0, CW)] + z1_v[2 * i + 1, pl.ds(0, CW)] + 1.0
      )
      for u in range(4):
        pair_v[i, pl.ds(u * CW, CW)] = v0
      for u in range(4, 8):
        pair_v[i, pl.ds(u * CW, CW)] = v1
      return 0

    lax.fori_loop(0, nrows, fill, 0)
    pltpu.sync_copy(
        pair_v.at[pl.ds(0, nrows)], out_ref.at[pl.ds(pr0 + local_off, nrows)]
    )

  emit(0, half)

  @pl.when(wid < NC * NT - 1)
  def _():
    emit(half, half)

  @pl.when(wid == NC * NT - 1)
  def _():
    emit(half, tail_rem)


def _cntpair_call(cparts):
  kern = pl.kernel(
      _cntpair_body,
      out_type=jax.ShapeDtypeStruct((NP, PW), jnp.float32),
      mesh=plsc.VectorSubcoreMesh(**_MESH),
      compiler_params=pltpu.CompilerParams(use_tc_tiling_on_sc=False),
      scratch_types=[
          pltpu.VMEM((2 * 392, CW), jnp.float32),
          pltpu.VMEM((2 * 392, CW), jnp.float32),
          pltpu.VMEM((392, PW), jnp.float32),
      ],
  )
  return kern(cparts)


RC = 1000


def _reformat_body(table_ref, tpart_ref, part_v0, part_v1, sem0, sem1):
  cid = lax.axis_index("c")
  sid = lax.axis_index("s")
  wid = cid * NT + sid
  zone = 1568
  row0 = wid * zone
  tail = N - 31 * zone
  part_v = (part_v0, part_v1)
  sems = (sem0, sem1)

  def emit(q, nrows):
    b = q % 2
    pltpu.make_async_copy(
        table_ref.at[pl.ds(row0, nrows), pl.ds(q * HP, HP)],
        part_v[b].at[pl.ds(0, nrows)],
        sems[b],
    ).start()

  def drain(q, nrows):
    b = q % 2
    pltpu.make_async_copy(
        table_ref.at[pl.ds(row0, nrows), pl.ds(q * HP, HP)],
        part_v[b].at[pl.ds(0, nrows)],
        sems[b],
    ).wait()
    pltpu.sync_copy(
        part_v[b].at[pl.ds(0, nrows)],
        tpart_ref.at[q].at[pl.ds(row0, nrows)],
    )

  def go(nrows):
    emit(0, nrows)
    emit(1, nrows)
    for q in range(NPARTS):
      drain(q, nrows)
      if q + 2 < NPARTS:
        emit(q + 2, nrows)

  @pl.when(wid < NC * NT - 1)
  def _():
    go(zone)

  @pl.when(wid == NC * NT - 1)
  def _():
    go(tail)


def _reformat_call(table):
  kern = pl.kernel(
      _reformat_body,
      out_type=jax.ShapeDtypeStruct((NPARTS, N, HP), jnp.float32),
      mesh=plsc.VectorSubcoreMesh(**_MESH),
      compiler_params=pltpu.CompilerParams(use_tc_tiling_on_sc=False),
      scratch_types=[
          pltpu.VMEM((1568, HP), jnp.float32),
          pltpu.VMEM((1568, HP), jnp.float32),
          pltpu.SemaphoreType.DMA,
          pltpu.SemaphoreType.DMA,
      ],
  )
  return kern(table)


def _conv_body(tpart_ref, src_ref, dst_ref, out_ref, src_v0, src_v1, dst_v0,
               dst_v1, rows_v0, rows_v1, sem_i0, sem_i1, sem_g, sem_s0,
               sem_s1, accum):
  cid = lax.axis_index("c")
  sid = lax.axis_index("s")
  row0 = sid * ROWS_MAIN
  src_v = (src_v0, src_v1)
  dst_v = (dst_v0, dst_v1)
  rows_v = (rows_v0, rows_v1)
  sem_i = (sem_i0, sem_i1)
  sem_s = (sem_s0, sem_s1)

  def idx_start(j, b):
    base = sid * EDGES_PER_TILE + j * KE
    pltpu.make_async_copy(
        src_ref.at[pl.ds(base, KE)], src_v[b], sem_i[b]
    ).start()
    pltpu.make_async_copy(
        dst_ref.at[pl.ds(base, KE)], dst_v[b], sem_i[b]
    ).start()

  def idx_wait(j, b):
    base = sid * EDGES_PER_TILE + j * KE
    pltpu.make_async_copy(
        src_ref.at[pl.ds(base, KE)], src_v[b], sem_i[b]
    ).wait()
    pltpu.make_async_copy(
        dst_ref.at[pl.ds(base, KE)], dst_v[b], sem_i[b]
    ).wait()

  for p in range(PASSES):
    part = cid * PASSES + p
    col0 = part * HP
    _zero_vmem_rows(rows_v0, KE, HP)
    pltpu.sync_copy(rows_v0, accum.at[pl.ds(row0, KE)])

    @pl.when(sid < NT - 1)
    def _():
      pltpu.sync_copy(
          rows_v0.at[pl.ds(0, ROWS_MAIN - KE)],
          accum.at[pl.ds(row0 + KE, ROWS_MAIN - KE)],
      )

    @pl.when(sid == NT - 1)
    def _():
      pltpu.sync_copy(
          rows_v0.at[pl.ds(0, ROWS_TAIL - KE)],
          accum.at[pl.ds(row0 + KE, ROWS_TAIL - KE)],
      )

    plsc.subcore_barrier()

    idx_start(0, 0)

    def step(j, _):
      for b in range(2):

        @pl.when(j % 2 == b)
        def _():
          nb = 1 - b
          idx_wait(j, b)
          pltpu.async_copy(
              tpart_ref.at[part].at[src_v[b]], rows_v[b], sem_g
          ).wait()

          @pl.when(j >= 1)
          def _():
            pltpu.make_async_copy(
                rows_v[nb], accum.at[dst_v[nb]], sem_s[nb]
            ).wait()

          @pl.when(j < CONV_CHUNKS - 1)
          def _():
            idx_start(j + 1, nb)

          pltpu.make_async_copy(
              rows_v[b], accum.at[dst_v[b]], sem_s[b]
          ).start(add=True)

      return 0

    lax.fori_loop(0, CONV_CHUNKS, step, 0)
    lastb = (CONV_CHUNKS - 1) % 2
    pltpu.make_async_copy(
        rows_v[lastb], accum.at[dst_v[lastb]], sem_s[lastb]
    ).wait()
    plsc.subcore_barrier()

    @pl.when(sid < NT - 1)
    def _():
      pltpu.sync_copy(
          accum.at[pl.ds(row0, ROWS_MAIN)],
          out_ref.at[pl.ds(row0, ROWS_MAIN), pl.ds(col0, HP)],
      )

    @pl.when(sid == NT - 1)
    def _():
      pltpu.sync_copy(
          accum.at[pl.ds(row0, ROWS_TAIL)],
          out_ref.at[pl.ds(row0, ROWS_TAIL), pl.ds(col0, HP)],
      )

    if p != PASSES - 1:
      plsc.subcore_barrier()


def _conv_call(table, src, dst):
  tpart = _reformat_call(table)
  kern = pl.kernel(
      _conv_body,
      out_type=jax.ShapeDtypeStruct((N, HID), jnp.float32),
      mesh=plsc.VectorSubcoreMesh(**_MESH),
      compiler_params=pltpu.CompilerParams(use_tc_tiling_on_sc=False),
      scratch_types=[
          pltpu.VMEM((KE,), jnp.int32),
          pltpu.VMEM((KE,), jnp.int32),
          pltpu.VMEM((KE,), jnp.int32),
          pltpu.VMEM((KE,), jnp.int32),
          pltpu.VMEM((KE, HP), jnp.float32),
          pltpu.VMEM((KE, HP), jnp.float32),
          pltpu.SemaphoreType.DMA,
          pltpu.SemaphoreType.DMA,
          pltpu.SemaphoreType.DMA,
          pltpu.SemaphoreType.DMA,
          pltpu.SemaphoreType.DMA,
          pltpu.VMEM_SHARED((N, CW), jnp.float32),
      ],
  )
  return kern(tpart, src, dst)


BP = 1000
NBLK = NP // BP


def _encoder_kernel(x_ref, wenc_ref, benc_ref, a1_ref, c1_ref, out_ref):
  r = jnp.maximum(
      jnp.dot(x_ref[...], wenc_ref[...], preferred_element_type=jnp.float32)
      + benc_ref[...],
      0.0,
  )
  out_ref[...] = (
      jnp.dot(r, a1_ref[...], preferred_element_type=jnp.float32) + c1_ref[...]
  )


def _encoder_call(x_pair, wenc2, benc2, a1d, c1d):
  return pl.pallas_call(
      _encoder_kernel,
      grid=(NBLK,),
      in_specs=[
          pl.BlockSpec((BP, 2 * IN_DIM), lambda i: (i, 0)),
          pl.BlockSpec((2 * IN_DIM, PW), lambda i: (0, 0)),
          pl.BlockSpec((1, PW), lambda i: (0, 0)),
          pl.BlockSpec((PW, PW), lambda i: (0, 0)),
          pl.BlockSpec((1, PW), lambda i: (0, 0)),
      ],
      out_specs=pl.BlockSpec((BP, PW), lambda i: (i, 0)),
      out_shape=jax.ShapeDtypeStruct((NP, PW), jnp.float32),
  )(x_pair, wenc2, benc2, a1d, c1d)


def _meanstats_kernel(s_ref, t_ref, cnt_ref, a_ref, stats_ref):
  i = pl.program_id(0)
  m = (s_ref[...] + t_ref[...]) / cnt_ref[...]
  a_ref[...] = m
  part = jnp.concatenate(
      [
          jnp.sum(m, axis=0, keepdims=True),
          jnp.sum(m * m, axis=0, keepdims=True),
      ],
      axis=0,
  )

  @pl.when(i == 0)
  def _():
    stats_ref[...] = part

  @pl.when(i > 0)
  def _():
    stats_ref[...] += part


def _meanstats_call(s_pair, t_pair, cnt_pair):
  return pl.pallas_call(
      _meanstats_kernel,
      grid=(NBLK,),
      in_specs=[
          pl.BlockSpec((BP, PW), lambda i: (i, 0)),
          pl.BlockSpec((BP, PW), lambda i: (i, 0)),
          pl.BlockSpec((BP, PW), lambda i: (i, 0)),
      ],
      out_specs=[
          pl.BlockSpec((BP, PW), lambda i: (i, 0)),
          pl.BlockSpec((2, PW), lambda i: (0, 0)),
      ],
      out_shape=[
          jax.ShapeDtypeStruct((NP, PW), jnp.float32),
          jax.ShapeDtypeStruct((2, PW), jnp.float32),
      ],
  )(s_pair, t_pair, cnt_pair)


def _bnmat_kernel(a_ref, stats_ref, w2_ref, out_ref):
  mean = (stats_ref[0:1, 0:HID] + stats_ref[0:1, HID:PW]) / N
  msq = (stats_ref[1:2, 0:HID] + stats_ref[1:2, HID:PW]) / N
  var = jnp.maximum(msq - mean * mean, 0.0)
  scale = lax.rsqrt(var + EPS)
  mean2 = jnp.concatenate([mean, mean], axis=1)
  scale2 = jnp.concatenate([scale, scale], axis=1)
  h = jnp.maximum((a_ref[...] - mean2) * scale2, 0.0)
  out_ref[...] = jnp.dot(h, w2_ref[...], preferred_element_type=jnp.float32)


def _bnmat_call(a_pair, stats, w2d):
  return pl.pallas_call(
      _bnmat_kernel,
      grid=(NBLK,),
      in_specs=[
          pl.BlockSpec((BP, PW), lambda i: (i, 0)),
          pl.BlockSpec((2, PW), lambda i: (0, 0)),
          pl.BlockSpec((PW, PW), lambda i: (0, 0)),
      ],
      out_specs=pl.BlockSpec((BP, PW), lambda i: (i, 0)),
      out_shape=jax.ShapeDtypeStruct((NP, PW), jnp.float32),
  )(a_pair, stats, w2d)


def _pool_kernel(s_ref, t_ref, cnt_ref, be_ref, bo_ref, b2_ref, wc1_ref,
                 bc1_ref, wc2_ref, bc2_ref, out_ref, acc_ref):
  i = pl.program_id(0)
  h = (s_ref[...] + t_ref[...]) / cnt_ref[...]
  be = jnp.reshape(be_ref[0], (1, BP))
  bo = jnp.reshape(bo_ref[0], (1, BP))
  giota = lax.broadcasted_iota(jnp.int32, (G, BP), 0)
  ohe = (giota == be).astype(jnp.float32)
  oho = (giota == bo).astype(jnp.float32)
  ones = jnp.ones((BP, HID), jnp.float32)
  he = jnp.concatenate([h[:, 0:HID], ones], axis=1)
  ho = jnp.concatenate([h[:, HID:PW], ones], axis=1)
  part = (
      jnp.dot(ohe, he, preferred_element_type=jnp.float32)
      + jnp.dot(oho, ho, preferred_element_type=jnp.float32)
  )

  @pl.when(i == 0)
  def _():
    acc_ref[...] = part

  @pl.when(i > 0)
  def _():
    acc_ref[...] += part

  @pl.when(i == NBLK - 1)
  def _():
    sums = acc_ref[:, 0:HID]
    gcnt = acc_ref[:, HID:HID + 1]
    pm = sums / jnp.maximum(gcnt, 1.0)
    pm = pm + jnp.where(gcnt > 0.0, 1.0, 0.0) * b2_ref[...]
    z = jnp.maximum(
        jnp.dot(pm, wc1_ref[...], preferred_element_type=jnp.float32)
        + bc1_ref[...],
        0.0,
    )
    out_ref[...] = (
        jnp.dot(z, wc2_ref[...], preferred_element_type=jnp.float32)
        + bc2_ref[...]
    )


def _pool_call(s_pair, t_pair, cnt_pair, batch_e, batch_o, b2, Wc1T, bc1,
               Wc2T, bc2):
  return pl.pallas_call(
      _pool_kernel,
      grid=(NBLK,),
      in_specs=[
          pl.BlockSpec((BP, PW), lambda i: (i, 0)),
          pl.BlockSpec((BP, PW), lambda i: (i, 0)),
          pl.BlockSpec((BP, PW), lambda i: (i, 0)),
          pl.BlockSpec((1, 1, BP), lambda i: (i, 0, 0)),
          pl.BlockSpec((1, 1, BP), lambda i: (i, 0, 0)),
          pl.BlockSpec((1, HID), lambda i: (0, 0)),
          pl.BlockSpec((HID, HID), lambda i: (0, 0)),
          pl.BlockSpec((1, HID), lambda i: (0, 0)),
          pl.BlockSpec((HID, OUT_DIM), lambda i: (0, 0)),
          pl.BlockSpec((1, OUT_DIM), lambda i: (0, 0)),
      ],
      out_specs=pl.BlockSpec((G, OUT_DIM), lambda i: (0, 0)),
      out_shape=jax.ShapeDtypeStruct((G, OUT_DIM), jnp.float32),
      scratch_shapes=[pltpu.VMEM((G, 2 * HID), jnp.float32)],
  )(s_pair, t_pair, cnt_pair, batch_e, batch_o, b2, Wc1T, bc1, Wc2T, bc2)


def _blockdiag(w):
  z = jnp.zeros_like(w)
  return jnp.concatenate(
      [jnp.concatenate([w, z], axis=1), jnp.concatenate([z, w], axis=1)],
      axis=0,
  )


def kernel(x, edge_index, batch, W_enc, b_enc, bn_gamma, bn_beta,
           W1, b1, W2, b2, Wc1, bc1, Wc2, bc2):
  g = bn_gamma / jnp.sqrt(1.0 + EPS)
  A1 = g[:, None] * W1.T
  c1 = bn_beta @ W1.T
  src = edge_index[0]
  dst = edge_index[1]

  x_pair = x.reshape(NP, 2 * IN_DIM)
  wenc2 = _blockdiag(W_enc.T)
  benc2 = jnp.tile(b_enc, 2)[None, :]
  a1d = _blockdiag(A1)
  c1d = jnp.tile(c1, 2)[None, :]
  w2d = _blockdiag(W2.T)
  batch_e = batch[0::2].reshape(NBLK, 1, BP)
  batch_o = batch[1::2].reshape(NBLK, 1, BP)

  t1_pair = _encoder_call(x_pair, wenc2, benc2, a1d, c1d)
  cnt_pair = _cntpair_call(_countacc_call(dst))
  s1_pair = _conv_call(t1_pair.reshape(N, HID), src, dst).reshape(NP, PW)
  a1_pair, stats = _meanstats_call(s1_pair, t1_pair, cnt_pair)
  t2_pair = _bnmat_call(a1_pair, stats, w2d)
  s2_pair = _conv_call(t2_pair.reshape(N, HID), src, dst).reshape(NP, PW)
  out = _pool_call(s2_pair, t2_pair, cnt_pair, batch_e, batch_o, b2[None, :],
                   Wc1.T, bc1[None, :], Wc2.T, bc2[None, :])
  return out


# speedup vs baseline: 20.0426x; 1.2273x over previous
import jax
import jax.numpy as jnp
from jax import lax
from jax.experimental import pallas as pl
from jax.experimental.pallas import tpu as pltpu
from jax.experimental.pallas import tpu_sc as plsc

N = 50000
E = 800000
IN_DIM = 128
HID = 64
OUT_DIM = 2
G = 64
EPS = 1e-5

NC = 2
NT = 16
NPARTS = 2
HP = HID // NPARTS
NP = N // 2
PW = 2 * HID

ROWS_MAIN = 3136
ROWS_TAIL = N - (NT - 1) * ROWS_MAIN

KE = 2000
EDGES_PER_TILE = E // NT
CONV_CHUNKS = EDGES_PER_TILE // KE

KC = 1000
EDGES_PER_WORKER = E // (NC * NT)
CNT_CHUNKS = EDGES_PER_WORKER // KC
CW = 16

_MESH = dict(core_axis_name="c", subcore_axis_name="s")


def _zero_vmem_rows(ref, nrows, width):
  zv = jnp.zeros((16,), jnp.float32)

  def body(i, _):
    for off in range(0, width, 16):
      ref[i, pl.ds(off, 16)] = zv
    return 0

  lax.fori_loop(0, nrows, body, 0)


def _fill_vmem_rows(ref, nrows, width, value):
  vv = jnp.full((16,), value, jnp.float32)

  def body(i, _):
    for off in range(0, width, 16):
      ref[i, pl.ds(off, 16)] = vv
    return 0

  lax.fori_loop(0, nrows, body, 0)


def _countacc_body(dst_ref, out_ref, idx_v, ones_v, zrows_v, accum):
  cid = lax.axis_index("c")
  sid = lax.axis_index("s")
  _zero_vmem_rows(zrows_v, ROWS_MAIN, CW)
  _fill_vmem_rows(ones_v, KC, CW, 1.0)
  row0 = sid * ROWS_MAIN

  @pl.when(sid < NT - 1)
  def _():
    pltpu.sync_copy(zrows_v, accum.at[pl.ds(row0, ROWS_MAIN)])

  @pl.when(sid == NT - 1)
  def _():
    pltpu.sync_copy(
        zrows_v.at[pl.ds(0, ROWS_TAIL)], accum.at[pl.ds(row0, ROWS_TAIL)]
    )

  plsc.subcore_barrier()

  wid = cid * NT + sid

  def step(j, _):
    base = wid * EDGES_PER_WORKER + j * KC
    pltpu.sync_copy(dst_ref.at[pl.ds(base, KC)], idx_v)
    pltpu.sync_copy(ones_v, accum.at[idx_v], add=True)
    return 0

  lax.fori_loop(0, CNT_CHUNKS, step, 0)
  plsc.subcore_barrier()

  @pl.when(sid < NT - 1)
  def _():
    pltpu.sync_copy(
        accum.at[pl.ds(row0, ROWS_MAIN)],
        out_ref.at[cid].at[pl.ds(row0, ROWS_MAIN)],
    )

  @pl.when(sid == NT - 1)
  def _():
    pltpu.sync_copy(
        accum.at[pl.ds(row0, ROWS_TAIL)],
        out_ref.at[cid].at[pl.ds(row0, ROWS_TAIL)],
    )


def _countacc_call(dst):
  kern = pl.kernel(
      _countacc_body,
      out_type=jax.ShapeDtypeStruct((NC, N, CW), jnp.float32),
      mesh=plsc.VectorSubcoreMesh(**_MESH),
      compiler_params=pltpu.CompilerParams(use_tc_tiling_on_sc=False),
      scratch_types=[
          pltpu.VMEM((KC,), jnp.int32),
          pltpu.VMEM((KC, CW), jnp.float32),
          pltpu.VMEM((ROWS_MAIN, CW), jnp.float32),
          pltpu.VMEM_SHARED((N, CW), jnp.float32),
      ],
  )
  return kern(dst)


def _cntpair_body(cp_ref, out_ref, z0_v, z1_v, pair_v):
  cid = lax.axis_index("c")
  sid = lax.axis_index("s")
  wid = cid * NT + sid
  zone = 784
  half = 392
  pr0 = wid * zone
  tail_rem = NP - 31 * zone - half

  def emit(local_off, nrows):
    n0 = 2 * (pr0 + local_off)
    pltpu.sync_copy(cp_ref.at[0].at[pl.ds(n0, 2 * nrows)],
                    z0_v.at[pl.ds(0, 2 * nrows)])
    pltpu.sync_copy(cp_ref.at[1].at[pl.ds(n0, 2 * nrows)],
                    z1_v.at[pl.ds(0, 2 * nrows)])

    def fill(i, _):
      v0 = z0_v[2 * i, pl.ds(0, CW)] + z1_v[2 * i, pl.ds(0, CW)] + 1.0
      v1 = (
          z0_v[2 * i + 1, pl.ds(0, CW)] + z1_v[2 * i + 1, pl.ds(0, CW)] + 1.0
      )
      for u in range(4):
        pair_v[i, pl.ds(u * CW, CW)] = v0
      for u in range(4, 8):
        pair_v[i, pl.ds(u * CW, CW)] = v1
      return 0

    lax.fori_loop(0, nrows, fill, 0)
    pltpu.sync_copy(
        pair_v.at[pl.ds(0, nrows)], out_ref.at[pl.ds(pr0 + local_off, nrows)]
    )

  emit(0, half)

  @pl.when(wid < NC * NT - 1)
  def _():
    emit(half, half)

  @pl.when(wid == NC * NT - 1)
  def _():
    emit(half, tail_rem)


def _cntpair_call(cparts):
  kern = pl.kernel(
      _cntpair_body,
      out_type=jax.ShapeDtypeStruct((NP, PW), jnp.float32),
      mesh=plsc.VectorSubcoreMesh(**_MESH),
      compiler_params=pltpu.CompilerParams(use_tc_tiling_on_sc=False),
      scratch_types=[
          pltpu.VMEM((2 * 392, CW), jnp.float32),
          pltpu.VMEM((2 * 392, CW), jnp.float32),
          pltpu.VMEM((392, PW), jnp.float32),
      ],
  )
  return kern(cparts)


RC = 1000


def _reformat_body(table_ref, tpart_ref, part_v0, part_v1, sem0, sem1):
  cid = lax.axis_index("c")
  sid = lax.axis_index("s")
  wid = cid * NT + sid
  zone = 1568
  row0 = wid * zone
  tail = N - 31 * zone
  part_v = (part_v0, part_v1)
  sems = (sem0, sem1)

  def emit(q, nrows):
    b = q % 2
    pltpu.make_async_copy(
        table_ref.at[pl.ds(row0, nrows), pl.ds(q * HP, HP)],
        part_v[b].at[pl.ds(0, nrows)],
        sems[b],
    ).start()

  def drain(q, nrows):
    b = q % 2
    pltpu.make_async_copy(
        table_ref.at[pl.ds(row0, nrows), pl.ds(q * HP, HP)],
        part_v[b].at[pl.ds(0, nrows)],
        sems[b],
    ).wait()
    pltpu.sync_copy(
        part_v[b].at[pl.ds(0, nrows)],
        tpart_ref.at[q].at[pl.ds(row0, nrows)],
    )

  def go(nrows):
    emit(0, nrows)
    emit(1, nrows)
    for q in range(NPARTS):
      drain(q, nrows)
      if q + 2 < NPARTS:
        emit(q + 2, nrows)

  @pl.when(wid < NC * NT - 1)
  def _():
    go(zone)

  @pl.when(wid == NC * NT - 1)
  def _():
    go(tail)


def _reformat_call(table):
  kern = pl.kernel(
      _reformat_body,
      out_type=jax.ShapeDtypeStruct((NPARTS, N, HP), jnp.bfloat16),
      mesh=plsc.VectorSubcoreMesh(**_MESH),
      compiler_params=pltpu.CompilerParams(use_tc_tiling_on_sc=False),
      scratch_types=[
          pltpu.VMEM((1568, HP), jnp.bfloat16),
          pltpu.VMEM((1568, HP), jnp.bfloat16),
          pltpu.SemaphoreType.DMA,
          pltpu.SemaphoreType.DMA,
      ],
  )
  return kern(table)


def _conv_body(tpart_ref, src_ref, dst_ref, out_ref, src_v0, src_v1, dst_v0,
               dst_v1, rows_v0, rows_v1, sem_i0, sem_i1, sem_g, sem_s0,
               sem_s1, accum):
  cid = lax.axis_index("c")
  sid = lax.axis_index("s")
  row0 = sid * ROWS_MAIN
  src_v = (src_v0, src_v1)
  dst_v = (dst_v0, dst_v1)
  rows_v = (rows_v0, rows_v1)
  sem_i = (sem_i0, sem_i1)
  sem_s = (sem_s0, sem_s1)

  def idx_start(j, b):
    base = sid * EDGES_PER_TILE + j * KE
    pltpu.make_async_copy(
        src_ref.at[pl.ds(base, KE)], src_v[b], sem_i[b]
    ).start()
    pltpu.make_async_copy(
        dst_ref.at[pl.ds(base, KE)], dst_v[b], sem_i[b]
    ).start()

  def idx_wait(j, b):
    base = sid * EDGES_PER_TILE + j * KE
    pltpu.make_async_copy(
        src_ref.at[pl.ds(base, KE)], src_v[b], sem_i[b]
    ).wait()
    pltpu.make_async_copy(
        dst_ref.at[pl.ds(base, KE)], dst_v[b], sem_i[b]
    ).wait()

  part = cid
  col0 = part * HP
  zb = jnp.zeros((32,), jnp.bfloat16)

  def zrow(i, _):
    rows_v0[i, pl.ds(0, HP)] = zb
    return 0

  lax.fori_loop(0, KE, zrow, 0)
  pltpu.sync_copy(rows_v0, accum.at[pl.ds(row0, KE)])

  @pl.when(sid < NT - 1)
  def _():
    pltpu.sync_copy(
        rows_v0.at[pl.ds(0, ROWS_MAIN - KE)],
        accum.at[pl.ds(row0 + KE, ROWS_MAIN - KE)],
    )

  @pl.when(sid == NT - 1)
  def _():
    pltpu.sync_copy(
        rows_v0.at[pl.ds(0, ROWS_TAIL - KE)],
        accum.at[pl.ds(row0 + KE, ROWS_TAIL - KE)],
    )

  plsc.subcore_barrier()

  idx_start(0, 0)

  def step(j, _):
    for b in range(2):

      @pl.when(j % 2 == b)
      def _():
        nb = 1 - b
        idx_wait(j, b)
        pltpu.async_copy(
            tpart_ref.at[part].at[src_v[b]], rows_v[b], sem_g
        ).wait()

        @pl.when(j >= 1)
        def _():
          pltpu.make_async_copy(
              rows_v[nb], accum.at[dst_v[nb]], sem_s[nb]
          ).wait()

        @pl.when(j < CONV_CHUNKS - 1)
        def _():
          idx_start(j + 1, nb)

        pltpu.make_async_copy(
            rows_v[b], accum.at[dst_v[b]], sem_s[b]
        ).start(add=True)

    return 0

  lax.fori_loop(0, CONV_CHUNKS, step, 0)
  lastb = (CONV_CHUNKS - 1) % 2
  pltpu.make_async_copy(
      rows_v[lastb], accum.at[dst_v[lastb]], sem_s[lastb]
  ).wait()
  plsc.subcore_barrier()

  @pl.when(sid < NT - 1)
  def _():
    pltpu.sync_copy(
        accum.at[pl.ds(row0, ROWS_MAIN)],
        out_ref.at[pl.ds(row0, ROWS_MAIN), pl.ds(col0, HP)],
    )

  @pl.when(sid == NT - 1)
  def _():
    pltpu.sync_copy(
        accum.at[pl.ds(row0, ROWS_TAIL)],
        out_ref.at[pl.ds(row0, ROWS_TAIL), pl.ds(col0, HP)],
    )


def _conv_call(table, src, dst):
  tpart = _reformat_call(table)
  kern = pl.kernel(
      _conv_body,
      out_type=jax.ShapeDtypeStruct((N, HID), jnp.bfloat16),
      mesh=plsc.VectorSubcoreMesh(**_MESH),
      compiler_params=pltpu.CompilerParams(use_tc_tiling_on_sc=False),
      scratch_types=[
          pltpu.VMEM((KE,), jnp.int32),
          pltpu.VMEM((KE,), jnp.int32),
          pltpu.VMEM((KE,), jnp.int32),
          pltpu.VMEM((KE,), jnp.int32),
          pltpu.VMEM((KE, HP), jnp.bfloat16),
          pltpu.VMEM((KE, HP), jnp.bfloat16),
          pltpu.SemaphoreType.DMA,
          pltpu.SemaphoreType.DMA,
          pltpu.SemaphoreType.DMA,
          pltpu.SemaphoreType.DMA,
          pltpu.SemaphoreType.DMA,
          pltpu.VMEM_SHARED((N, HP), jnp.bfloat16),
      ],
  )
  return kern(tpart, src, dst)


BP = 1000
NBLK = NP // BP


def _encoder_kernel(x_ref, wenc_ref, benc_ref, a1_ref, c1_ref, out_ref):
  r = jnp.maximum(
      jnp.dot(x_ref[...], wenc_ref[...], preferred_element_type=jnp.float32)
      + benc_ref[...],
      0.0,
  )
  out_ref[...] = (
      jnp.dot(r, a1_ref[...], preferred_element_type=jnp.float32) + c1_ref[...]
  ).astype(jnp.bfloat16)


def _encoder_call(x_pair, wenc2, benc2, a1d, c1d):
  return pl.pallas_call(
      _encoder_kernel,
      grid=(NBLK,),
      in_specs=[
          pl.BlockSpec((BP, 2 * IN_DIM), lambda i: (i, 0)),
          pl.BlockSpec((2 * IN_DIM, PW), lambda i: (0, 0)),
          pl.BlockSpec((1, PW), lambda i: (0, 0)),
          pl.BlockSpec((PW, PW), lambda i: (0, 0)),
          pl.BlockSpec((1, PW), lambda i: (0, 0)),
      ],
      out_specs=pl.BlockSpec((BP, PW), lambda i: (i, 0)),
      out_shape=jax.ShapeDtypeStruct((NP, PW), jnp.bfloat16),
  )(x_pair, wenc2, benc2, a1d, c1d)


def _meanstats_kernel(s_ref, t_ref, cnt_ref, a_ref, stats_ref):
  i = pl.program_id(0)
  m = (
      s_ref[...].astype(jnp.float32) + t_ref[...].astype(jnp.float32)
  ) / cnt_ref[...]
  a_ref[...] = m
  part = jnp.concatenate(
      [
          jnp.sum(m, axis=0, keepdims=True),
          jnp.sum(m * m, axis=0, keepdims=True),
      ],
      axis=0,
  )

  @pl.when(i == 0)
  def _():
    stats_ref[...] = part

  @pl.when(i > 0)
  def _():
    stats_ref[...] += part


def _meanstats_call(s_pair, t_pair, cnt_pair):
  return pl.pallas_call(
      _meanstats_kernel,
      grid=(NBLK,),
      in_specs=[
          pl.BlockSpec((BP, PW), lambda i: (i, 0)),
          pl.BlockSpec((BP, PW), lambda i: (i, 0)),
          pl.BlockSpec((BP, PW), lambda i: (i, 0)),
      ],
      out_specs=[
          pl.BlockSpec((BP, PW), lambda i: (i, 0)),
          pl.BlockSpec((2, PW), lambda i: (0, 0)),
      ],
      out_shape=[
          jax.ShapeDtypeStruct((NP, PW), jnp.float32),
          jax.ShapeDtypeStruct((2, PW), jnp.float32),
      ],
  )(s_pair, t_pair, cnt_pair)


def _bnmat_kernel(a_ref, stats_ref, w2_ref, out_ref):
  mean = (stats_ref[0:1, 0:HID] + stats_ref[0:1, HID:PW]) / N
  msq = (stats_ref[1:2, 0:HID] + stats_ref[1:2, HID:PW]) / N
  var = jnp.maximum(msq - mean * mean, 0.0)
  scale = lax.rsqrt(var + EPS)
  mean2 = jnp.concatenate([mean, mean], axis=1)
  scale2 = jnp.concatenate([scale, scale], axis=1)
  h = jnp.maximum((a_ref[...] - mean2) * scale2, 0.0)
  out_ref[...] = jnp.dot(
      h, w2_ref[...], preferred_element_type=jnp.float32
  ).astype(jnp.bfloat16)


def _bnmat_call(a_pair, stats, w2d):
  return pl.pallas_call(
      _bnmat_kernel,
      grid=(NBLK,),
      in_specs=[
          pl.BlockSpec((BP, PW), lambda i: (i, 0)),
          pl.BlockSpec((2, PW), lambda i: (0, 0)),
          pl.BlockSpec((PW, PW), lambda i: (0, 0)),
      ],
      out_specs=pl.BlockSpec((BP, PW), lambda i: (i, 0)),
      out_shape=jax.ShapeDtypeStruct((NP, PW), jnp.bfloat16),
  )(a_pair, stats, w2d)


def _pool_kernel(s_ref, t_ref, cnt_ref, be_ref, bo_ref, b2_ref, wc1_ref,
                 bc1_ref, wc2_ref, bc2_ref, out_ref, acc_ref):
  i = pl.program_id(0)
  h = (
      s_ref[...].astype(jnp.float32) + t_ref[...].astype(jnp.float32)
  ) / cnt_ref[...]
  be = jnp.reshape(be_ref[0], (1, BP))
  bo = jnp.reshape(bo_ref[0], (1, BP))
  giota = lax.broadcasted_iota(jnp.int32, (G, BP), 0)
  ohe = (giota == be).astype(jnp.float32)
  oho = (giota == bo).astype(jnp.float32)
  ones = jnp.ones((BP, HID), jnp.float32)
  he = jnp.concatenate([h[:, 0:HID], ones], axis=1)
  ho = jnp.concatenate([h[:, HID:PW], ones], axis=1)
  part = (
      jnp.dot(ohe, he, preferred_element_type=jnp.float32)
      + jnp.dot(oho, ho, preferred_element_type=jnp.float32)
  )

  @pl.when(i == 0)
  def _():
    acc_ref[...] = part

  @pl.when(i > 0)
  def _():
    acc_ref[...] += part

  @pl.when(i == NBLK - 1)
  def _():
    sums = acc_ref[:, 0:HID]
    gcnt = acc_ref[:, HID:HID + 1]
    pm = sums / jnp.maximum(gcnt, 1.0)
    pm = pm + jnp.where(gcnt > 0.0, 1.0, 0.0) * b2_ref[...]
    z = jnp.maximum(
        jnp.dot(pm, wc1_ref[...], preferred_element_type=jnp.float32)
        + bc1_ref[...],
        0.0,
    )
    out_ref[...] = (
        jnp.dot(z, wc2_ref[...], preferred_element_type=jnp.float32)
        + bc2_ref[...]
    )


def _pool_call(s_pair, t_pair, cnt_pair, batch_e, batch_o, b2, Wc1T, bc1,
               Wc2T, bc2):
  return pl.pallas_call(
      _pool_kernel,
      grid=(NBLK,),
      in_specs=[
          pl.BlockSpec((BP, PW), lambda i: (i, 0)),
          pl.BlockSpec((BP, PW), lambda i: (i, 0)),
          pl.BlockSpec((BP, PW), lambda i: (i, 0)),
          pl.BlockSpec((1, 1, BP), lambda i: (i, 0, 0)),
          pl.BlockSpec((1, 1, BP), lambda i: (i, 0, 0)),
          pl.BlockSpec((1, HID), lambda i: (0, 0)),
          pl.BlockSpec((HID, HID), lambda i: (0, 0)),
          pl.BlockSpec((1, HID), lambda i: (0, 0)),
          pl.BlockSpec((HID, OUT_DIM), lambda i: (0, 0)),
          pl.BlockSpec((1, OUT_DIM), lambda i: (0, 0)),
      ],
      out_specs=pl.BlockSpec((G, OUT_DIM), lambda i: (0, 0)),
      out_shape=jax.ShapeDtypeStruct((G, OUT_DIM), jnp.float32),
      scratch_shapes=[pltpu.VMEM((G, 2 * HID), jnp.float32)],
  )(s_pair, t_pair, cnt_pair, batch_e, batch_o, b2, Wc1T, bc1, Wc2T, bc2)


def _blockdiag(w):
  z = jnp.zeros_like(w)
  return jnp.concatenate(
      [jnp.concatenate([w, z], axis=1), jnp.concatenate([z, w], axis=1)],
      axis=0,
  )


def kernel(x, edge_index, batch, W_enc, b_enc, bn_gamma, bn_beta,
           W1, b1, W2, b2, Wc1, bc1, Wc2, bc2):
  g = bn_gamma / jnp.sqrt(1.0 + EPS)
  A1 = g[:, None] * W1.T
  c1 = bn_beta @ W1.T
  src = edge_index[0]
  dst = edge_index[1]

  x_pair = x.reshape(NP, 2 * IN_DIM)
  wenc2 = _blockdiag(W_enc.T)
  benc2 = jnp.tile(b_enc, 2)[None, :]
  a1d = _blockdiag(A1)
  c1d = jnp.tile(c1, 2)[None, :]
  w2d = _blockdiag(W2.T)
  batch_e = batch[0::2].reshape(NBLK, 1, BP)
  batch_o = batch[1::2].reshape(NBLK, 1, BP)

  t1_pair = _encoder_call(x_pair, wenc2, benc2, a1d, c1d)
  cnt_pair = _cntpair_call(_countacc_call(dst))
  s1_pair = _conv_call(t1_pair.reshape(N, HID), src, dst).reshape(NP, PW)
  a1_pair, stats = _meanstats_call(s1_pair, t1_pair, cnt_pair)
  t2_pair = _bnmat_call(a1_pair, stats, w2d)
  s2_pair = _conv_call(t2_pair.reshape(N, HID), src, dst).reshape(NP, PW)
  out = _pool_call(s2_pair, t2_pair, cnt_pair, batch_e, batch_o, b2[None, :],
                   Wc1.T, bc1[None, :], Wc2.T, bc2[None, :])
  return out


# speedup vs baseline: 20.5525x; 1.0254x over previous
import jax
import jax.numpy as jnp
from jax import lax
from jax.experimental import pallas as pl
from jax.experimental.pallas import tpu as pltpu
from jax.experimental.pallas import tpu_sc as plsc

N = 50000
E = 800000
IN_DIM = 128
HID = 64
OUT_DIM = 2
G = 64
EPS = 1e-5

NC = 2
NT = 16
NPARTS = 2
HP = HID // NPARTS
NP = N // 2
PW = 2 * HID

ROWS_MAIN = 3136
ROWS_TAIL = N - (NT - 1) * ROWS_MAIN

KE = 2000
EDGES_PER_TILE = E // NT
CONV_CHUNKS = EDGES_PER_TILE // KE

KC = 1000
EDGES_PER_WORKER = E // (NC * NT)
CNT_CHUNKS = EDGES_PER_WORKER // KC
CW = 16

_MESH = dict(core_axis_name="c", subcore_axis_name="s")


def _zero_vmem_rows(ref, nrows, width):
  zv = jnp.zeros((16,), jnp.float32)

  def body(i, _):
    for off in range(0, width, 16):
      ref[i, pl.ds(off, 16)] = zv
    return 0

  lax.fori_loop(0, nrows, body, 0)


def _fill_vmem_rows(ref, nrows, width, value):
  vv = jnp.full((16,), value, jnp.float32)

  def body(i, _):
    for off in range(0, width, 16):
      ref[i, pl.ds(off, 16)] = vv
    return 0

  lax.fori_loop(0, nrows, body, 0)


def _countacc_body(dst_ref, out_ref, idx_v0, idx_v1, ones_v, zrows_v, sem_i0,
                   sem_i1, sem_s0, sem_s1, accum):
  cid = lax.axis_index("c")
  sid = lax.axis_index("s")
  _zero_vmem_rows(zrows_v, ROWS_MAIN, CW)
  _fill_vmem_rows(ones_v, KC, CW, 1.0)
  row0 = sid * ROWS_MAIN
  idx_v = (idx_v0, idx_v1)
  sem_i = (sem_i0, sem_i1)
  sem_s = (sem_s0, sem_s1)

  @pl.when(sid < NT - 1)
  def _():
    pltpu.sync_copy(zrows_v, accum.at[pl.ds(row0, ROWS_MAIN)])

  @pl.when(sid == NT - 1)
  def _():
    pltpu.sync_copy(
        zrows_v.at[pl.ds(0, ROWS_TAIL)], accum.at[pl.ds(row0, ROWS_TAIL)]
    )

  plsc.subcore_barrier()

  wid = cid * NT + sid

  def idx_start(j, b):
    base = wid * EDGES_PER_WORKER + j * KC
    pltpu.make_async_copy(
        dst_ref.at[pl.ds(base, KC)], idx_v[b], sem_i[b]
    ).start()

  def idx_wait(j, b):
    base = wid * EDGES_PER_WORKER + j * KC
    pltpu.make_async_copy(
        dst_ref.at[pl.ds(base, KC)], idx_v[b], sem_i[b]
    ).wait()

  idx_start(0, 0)

  def step(j, _):
    for b in range(2):

      @pl.when(j % 2 == b)
      def _():
        nb = 1 - b
        idx_wait(j, b)

        @pl.when(j >= 1)
        def _():
          pltpu.make_async_copy(
              ones_v, accum.at[idx_v[nb]], sem_s[nb]
          ).wait()

        @pl.when(j < CNT_CHUNKS - 1)
        def _():
          idx_start(j + 1, nb)

        pltpu.make_async_copy(
            ones_v, accum.at[idx_v[b]], sem_s[b]
        ).start(add=True)

    return 0

  lax.fori_loop(0, CNT_CHUNKS, step, 0)
  lastb = (CNT_CHUNKS - 1) % 2
  pltpu.make_async_copy(
      ones_v, accum.at[idx_v[lastb]], sem_s[lastb]
  ).wait()
  plsc.subcore_barrier()

  @pl.when(sid < NT - 1)
  def _():
    pltpu.sync_copy(
        accum.at[pl.ds(row0, ROWS_MAIN)],
        out_ref.at[cid].at[pl.ds(row0, ROWS_MAIN)],
    )

  @pl.when(sid == NT - 1)
  def _():
    pltpu.sync_copy(
        accum.at[pl.ds(row0, ROWS_TAIL)],
        out_ref.at[cid].at[pl.ds(row0, ROWS_TAIL)],
    )


def _countacc_call(dst):
  kern = pl.kernel(
      _countacc_body,
      out_type=jax.ShapeDtypeStruct((NC, N, CW), jnp.float32),
      mesh=plsc.VectorSubcoreMesh(**_MESH),
      compiler_params=pltpu.CompilerParams(use_tc_tiling_on_sc=False),
      scratch_types=[
          pltpu.VMEM((KC,), jnp.int32),
          pltpu.VMEM((KC,), jnp.int32),
          pltpu.VMEM((KC, CW), jnp.float32),
          pltpu.VMEM((ROWS_MAIN, CW), jnp.float32),
          pltpu.SemaphoreType.DMA,
          pltpu.SemaphoreType.DMA,
          pltpu.SemaphoreType.DMA,
          pltpu.SemaphoreType.DMA,
          pltpu.VMEM_SHARED((N, CW), jnp.float32),
      ],
  )
  return kern(dst)


def _cntpair_body(cp_ref, out_ref, z0_v, z1_v, pair_v):
  cid = lax.axis_index("c")
  sid = lax.axis_index("s")
  wid = cid * NT + sid
  zone = 784
  half = 392
  pr0 = wid * zone
  tail_rem = NP - 31 * zone - half

  def emit(local_off, nrows):
    n0 = 2 * (pr0 + local_off)
    pltpu.sync_copy(cp_ref.at[0].at[pl.ds(n0, 2 * nrows)],
                    z0_v.at[pl.ds(0, 2 * nrows)])
    pltpu.sync_copy(cp_ref.at[1].at[pl.ds(n0, 2 * nrows)],
                    z1_v.at[pl.ds(0, 2 * nrows)])

    def fill(i, _):
      v0 = z0_v[2 * i, pl.ds(0, CW)] + z1_v[2 * i, pl.ds(0, CW)] + 1.0
      v1 = (
          z0_v[2 * i + 1, pl.ds(0, CW)] + z1_v[2 * i + 1, pl.ds(0, CW)] + 1.0
      )
      for u in range(4):
        pair_v[i, pl.ds(u * CW, CW)] = v0
      for u in range(4, 8):
        pair_v[i, pl.ds(u * CW, CW)] = v1
      return 0

    lax.fori_loop(0, nrows, fill, 0)
    pltpu.sync_copy(
        pair_v.at[pl.ds(0, nrows)], out_ref.at[pl.ds(pr0 + local_off, nrows)]
    )

  emit(0, half)

  @pl.when(wid < NC * NT - 1)
  def _():
    emit(half, half)

  @pl.when(wid == NC * NT - 1)
  def _():
    emit(half, tail_rem)


def _cntpair_call(cparts):
  kern = pl.kernel(
      _cntpair_body,
      out_type=jax.ShapeDtypeStruct((NP, PW), jnp.float32),
      mesh=plsc.VectorSubcoreMesh(**_MESH),
      compiler_params=pltpu.CompilerParams(use_tc_tiling_on_sc=False),
      scratch_types=[
          pltpu.VMEM((2 * 392, CW), jnp.float32),
          pltpu.VMEM((2 * 392, CW), jnp.float32),
          pltpu.VMEM((392, PW), jnp.float32),
      ],
  )
  return kern(cparts)


RC = 1000


def _reformat_body(table_ref, tpart_ref, part_v0, part_v1, sem0, sem1):
  cid = lax.axis_index("c")
  sid = lax.axis_index("s")
  wid = cid * NT + sid
  zone = 1568
  row0 = wid * zone
  tail = N - 31 * zone
  part_v = (part_v0, part_v1)
  sems = (sem0, sem1)

  def emit(q, nrows):
    b = q % 2
    pltpu.make_async_copy(
        table_ref.at[pl.ds(row0, nrows), pl.ds(q * HP, HP)],
        part_v[b].at[pl.ds(0, nrows)],
        sems[b],
    ).start()

  def drain(q, nrows):
    b = q % 2
    pltpu.make_async_copy(
        table_ref.at[pl.ds(row0, nrows), pl.ds(q * HP, HP)],
        part_v[b].at[pl.ds(0, nrows)],
        sems[b],
    ).wait()
    pltpu.sync_copy(
        part_v[b].at[pl.ds(0, nrows)],
        tpart_ref.at[q].at[pl.ds(row0, nrows)],
    )

  def go(nrows):
    emit(0, nrows)
    emit(1, nrows)
    for q in range(NPARTS):
      drain(q, nrows)
      if q + 2 < NPARTS:
        emit(q + 2, nrows)

  @pl.when(wid < NC * NT - 1)
  def _():
    go(zone)

  @pl.when(wid == NC * NT - 1)
  def _():
    go(tail)


def _reformat_call(table):
  kern = pl.kernel(
      _reformat_body,
      out_type=jax.ShapeDtypeStruct((NPARTS, N, HP), jnp.bfloat16),
      mesh=plsc.VectorSubcoreMesh(**_MESH),
      compiler_params=pltpu.CompilerParams(use_tc_tiling_on_sc=False),
      scratch_types=[
          pltpu.VMEM((1568, HP), jnp.bfloat16),
          pltpu.VMEM((1568, HP), jnp.bfloat16),
          pltpu.SemaphoreType.DMA,
          pltpu.SemaphoreType.DMA,
      ],
  )
  return kern(table)


def _conv_body(tpart_ref, src_ref, dst_ref, out_ref, src_v0, src_v1, dst_v0,
               dst_v1, rows_v0, rows_v1, sem_i0, sem_i1, sem_g, sem_s0,
               sem_s1, accum):
  cid = lax.axis_index("c")
  sid = lax.axis_index("s")
  row0 = sid * ROWS_MAIN
  src_v = (src_v0, src_v1)
  dst_v = (dst_v0, dst_v1)
  rows_v = (rows_v0, rows_v1)
  sem_i = (sem_i0, sem_i1)
  sem_s = (sem_s0, sem_s1)

  def idx_start(j, b):
    base = sid * EDGES_PER_TILE + j * KE
    pltpu.make_async_copy(
        src_ref.at[pl.ds(base, KE)], src_v[b], sem_i[b]
    ).start()
    pltpu.make_async_copy(
        dst_ref.at[pl.ds(base, KE)], dst_v[b], sem_i[b]
    ).start()

  def idx_wait(j, b):
    base = sid * EDGES_PER_TILE + j * KE
    pltpu.make_async_copy(
        src_ref.at[pl.ds(base, KE)], src_v[b], sem_i[b]
    ).wait()
    pltpu.make_async_copy(
        dst_ref.at[pl.ds(base, KE)], dst_v[b], sem_i[b]
    ).wait()

  part = cid
  col0 = part * HP
  zb = jnp.zeros((32,), jnp.bfloat16)

  def zrow(i, _):
    rows_v0[i, pl.ds(0, HP)] = zb
    return 0

  lax.fori_loop(0, KE, zrow, 0)
  pltpu.sync_copy(rows_v0, accum.at[pl.ds(row0, KE)])

  @pl.when(sid < NT - 1)
  def _():
    pltpu.sync_copy(
        rows_v0.at[pl.ds(0, ROWS_MAIN - KE)],
        accum.at[pl.ds(row0 + KE, ROWS_MAIN - KE)],
    )

  @pl.when(sid == NT - 1)
  def _():
    pltpu.sync_copy(
        rows_v0.at[pl.ds(0, ROWS_TAIL - KE)],
        accum.at[pl.ds(row0 + KE, ROWS_TAIL - KE)],
    )

  plsc.subcore_barrier()

  idx_start(0, 0)

  def step(j, _):
    for b in range(2):

      @pl.when(j % 2 == b)
      def _():
        nb = 1 - b
        idx_wait(j, b)
        pltpu.async_copy(
            tpart_ref.at[part].at[src_v[b]], rows_v[b], sem_g
        ).wait()

        @pl.when(j >= 1)
        def _():
          pltpu.make_async_copy(
              rows_v[nb], accum.at[dst_v[nb]], sem_s[nb]
          ).wait()

        @pl.when(j < CONV_CHUNKS - 1)
        def _():
          idx_start(j + 1, nb)

        pltpu.make_async_copy(
            rows_v[b], accum.at[dst_v[b]], sem_s[b]
        ).start(add=True)

    return 0

  lax.fori_loop(0, CONV_CHUNKS, step, 0)
  lastb = (CONV_CHUNKS - 1) % 2
  pltpu.make_async_copy(
      rows_v[lastb], accum.at[dst_v[lastb]], sem_s[lastb]
  ).wait()
  plsc.subcore_barrier()

  @pl.when(sid < NT - 1)
  def _():
    pltpu.sync_copy(
        accum.at[pl.ds(row0, ROWS_MAIN)],
        out_ref.at[pl.ds(row0, ROWS_MAIN), pl.ds(col0, HP)],
    )

  @pl.when(sid == NT - 1)
  def _():
    pltpu.sync_copy(
        accum.at[pl.ds(row0, ROWS_TAIL)],
        out_ref.at[pl.ds(row0, ROWS_TAIL), pl.ds(col0, HP)],
    )


def _conv_call(table, src, dst):
  tpart = _reformat_call(table)
  kern = pl.kernel(
      _conv_body,
      out_type=jax.ShapeDtypeStruct((N, HID), jnp.bfloat16),
      mesh=plsc.VectorSubcoreMesh(**_MESH),
      compiler_params=pltpu.CompilerParams(use_tc_tiling_on_sc=False),
      scratch_types=[
          pltpu.VMEM((KE,), jnp.int32),
          pltpu.VMEM((KE,), jnp.int32),
          pltpu.VMEM((KE,), jnp.int32),
          pltpu.VMEM((KE,), jnp.int32),
          pltpu.VMEM((KE, HP), jnp.bfloat16),
          pltpu.VMEM((KE, HP), jnp.bfloat16),
          pltpu.SemaphoreType.DMA,
          pltpu.SemaphoreType.DMA,
          pltpu.SemaphoreType.DMA,
          pltpu.SemaphoreType.DMA,
          pltpu.SemaphoreType.DMA,
          pltpu.VMEM_SHARED((N, HP), jnp.bfloat16),
      ],
  )
  return kern(tpart, src, dst)


BP = 1000
NBLK = NP // BP


def _encoder_kernel(x_ref, wenc_ref, benc_ref, a1_ref, c1_ref, out_ref):
  r = jnp.maximum(
      jnp.dot(x_ref[...], wenc_ref[...], preferred_element_type=jnp.float32)
      + benc_ref[...],
      0.0,
  )
  out_ref[...] = (
      jnp.dot(r, a1_ref[...], preferred_element_type=jnp.float32) + c1_ref[...]
  ).astype(jnp.bfloat16)


def _encoder_call(x_pair, wenc2, benc2, a1d, c1d):
  return pl.pallas_call(
      _encoder_kernel,
      grid=(NBLK,),
      in_specs=[
          pl.BlockSpec((BP, 2 * IN_DIM), lambda i: (i, 0)),
          pl.BlockSpec((2 * IN_DIM, PW), lambda i: (0, 0)),
          pl.BlockSpec((1, PW), lambda i: (0, 0)),
          pl.BlockSpec((PW, PW), lambda i: (0, 0)),
          pl.BlockSpec((1, PW), lambda i: (0, 0)),
      ],
      out_specs=pl.BlockSpec((BP, PW), lambda i: (i, 0)),
      out_shape=jax.ShapeDtypeStruct((NP, PW), jnp.bfloat16),
  )(x_pair, wenc2, benc2, a1d, c1d)


def _meanstats_kernel(s_ref, t_ref, cnt_ref, a_ref, stats_ref):
  i = pl.program_id(0)
  m = (
      s_ref[...].astype(jnp.float32) + t_ref[...].astype(jnp.float32)
  ) / cnt_ref[...]
  a_ref[...] = m.astype(jnp.bfloat16)
  part = jnp.concatenate(
      [
          jnp.sum(m, axis=0, keepdims=True),
          jnp.sum(m * m, axis=0, keepdims=True),
      ],
      axis=0,
  )

  @pl.when(i == 0)
  def _():
    stats_ref[...] = part

  @pl.when(i > 0)
  def _():
    stats_ref[...] += part


def _meanstats_call(s_pair, t_pair, cnt_pair):
  return pl.pallas_call(
      _meanstats_kernel,
      grid=(NBLK,),
      in_specs=[
          pl.BlockSpec((BP, PW), lambda i: (i, 0)),
          pl.BlockSpec((BP, PW), lambda i: (i, 0)),
          pl.BlockSpec((BP, PW), lambda i: (i, 0)),
      ],
      out_specs=[
          pl.BlockSpec((BP, PW), lambda i: (i, 0)),
          pl.BlockSpec((2, PW), lambda i: (0, 0)),
      ],
      out_shape=[
          jax.ShapeDtypeStruct((NP, PW), jnp.bfloat16),
          jax.ShapeDtypeStruct((2, PW), jnp.float32),
      ],
  )(s_pair, t_pair, cnt_pair)


def _bnmat_kernel(a_ref, stats_ref, w2_ref, out_ref):
  mean = (stats_ref[0:1, 0:HID] + stats_ref[0:1, HID:PW]) / N
  msq = (stats_ref[1:2, 0:HID] + stats_ref[1:2, HID:PW]) / N
  var = jnp.maximum(msq - mean * mean, 0.0)
  scale = lax.rsqrt(var + EPS)
  mean2 = jnp.concatenate([mean, mean], axis=1)
  scale2 = jnp.concatenate([scale, scale], axis=1)
  h = jnp.maximum((a_ref[...].astype(jnp.float32) - mean2) * scale2, 0.0)
  out_ref[...] = jnp.dot(
      h, w2_ref[...], preferred_element_type=jnp.float32
  ).astype(jnp.bfloat16)


def _bnmat_call(a_pair, stats, w2d):
  return pl.pallas_call(
      _bnmat_kernel,
      grid=(NBLK,),
      in_specs=[
          pl.BlockSpec((BP, PW), lambda i: (i, 0)),
          pl.BlockSpec((2, PW), lambda i: (0, 0)),
          pl.BlockSpec((PW, PW), lambda i: (0, 0)),
      ],
      out_specs=pl.BlockSpec((BP, PW), lambda i: (i, 0)),
      out_shape=jax.ShapeDtypeStruct((NP, PW), jnp.bfloat16),
  )(a_pair, stats, w2d)


def _pool_kernel(s_ref, t_ref, cnt_ref, be_ref, bo_ref, b2_ref, wc1_ref,
                 bc1_ref, wc2_ref, bc2_ref, out_ref, acc_ref):
  i = pl.program_id(0)
  h = (
      s_ref[...].astype(jnp.float32) + t_ref[...].astype(jnp.float32)
  ) / cnt_ref[...]
  be = jnp.reshape(be_ref[0], (1, BP))
  bo = jnp.reshape(bo_ref[0], (1, BP))
  giota = lax.broadcasted_iota(jnp.int32, (G, BP), 0)
  ohe = (giota == be).astype(jnp.float32)
  oho = (giota == bo).astype(jnp.float32)
  ones = jnp.ones((BP, HID), jnp.float32)
  he = jnp.concatenate([h[:, 0:HID], ones], axis=1)
  ho = jnp.concatenate([h[:, HID:PW], ones], axis=1)
  part = (
      jnp.dot(ohe, he, preferred_element_type=jnp.float32)
      + jnp.dot(oho, ho, preferred_element_type=jnp.float32)
  )

  @pl.when(i == 0)
  def _():
    acc_ref[...] = part

  @pl.when(i > 0)
  def _():
    acc_ref[...] += part

  @pl.when(i == NBLK - 1)
  def _():
    sums = acc_ref[:, 0:HID]
    gcnt = acc_ref[:, HID:HID + 1]
    pm = sums / jnp.maximum(gcnt, 1.0)
    pm = pm + jnp.where(gcnt > 0.0, 1.0, 0.0) * b2_ref[...]
    z = jnp.maximum(
        jnp.dot(pm, wc1_ref[...], preferred_element_type=jnp.float32)
        + bc1_ref[...],
        0.0,
    )
    out_ref[...] = (
        jnp.dot(z, wc2_ref[...], preferred_element_type=jnp.float32)
        + bc2_ref[...]
    )


def _pool_call(s_pair, t_pair, cnt_pair, batch_e, batch_o, b2, Wc1T, bc1,
               Wc2T, bc2):
  return pl.pallas_call(
      _pool_kernel,
      grid=(NBLK,),
      in_specs=[
          pl.BlockSpec((BP, PW), lambda i: (i, 0)),
          pl.BlockSpec((BP, PW), lambda i: (i, 0)),
          pl.BlockSpec((BP, PW), lambda i: (i, 0)),
          pl.BlockSpec((1, 1, BP), lambda i: (i, 0, 0)),
          pl.BlockSpec((1, 1, BP), lambda i: (i, 0, 0)),
          pl.BlockSpec((1, HID), lambda i: (0, 0)),
          pl.BlockSpec((HID, HID), lambda i: (0, 0)),
          pl.BlockSpec((1, HID), lambda i: (0, 0)),
          pl.BlockSpec((HID, OUT_DIM), lambda i: (0, 0)),
          pl.BlockSpec((1, OUT_DIM), lambda i: (0, 0)),
      ],
      out_specs=pl.BlockSpec((G, OUT_DIM), lambda i: (0, 0)),
      out_shape=jax.ShapeDtypeStruct((G, OUT_DIM), jnp.float32),
      scratch_shapes=[pltpu.VMEM((G, 2 * HID), jnp.float32)],
  )(s_pair, t_pair, cnt_pair, batch_e, batch_o, b2, Wc1T, bc1, Wc2T, bc2)


def _blockdiag(w):
  z = jnp.zeros_like(w)
  return jnp.concatenate(
      [jnp.concatenate([w, z], axis=1), jnp.concatenate([z, w], axis=1)],
      axis=0,
  )


def kernel(x, edge_index, batch, W_enc, b_enc, bn_gamma, bn_beta,
           W1, b1, W2, b2, Wc1, bc1, Wc2, bc2):
  g = bn_gamma / jnp.sqrt(1.0 + EPS)
  A1 = g[:, None] * W1.T
  c1 = bn_beta @ W1.T
  src = edge_index[0]
  dst = edge_index[1]

  x_pair = x.reshape(NP, 2 * IN_DIM)
  wenc2 = _blockdiag(W_enc.T)
  benc2 = jnp.tile(b_enc, 2)[None, :]
  a1d = _blockdiag(A1)
  c1d = jnp.tile(c1, 2)[None, :]
  w2d = _blockdiag(W2.T)
  batch_e = batch[0::2].reshape(NBLK, 1, BP)
  batch_o = batch[1::2].reshape(NBLK, 1, BP)

  t1_pair = _encoder_call(x_pair, wenc2, benc2, a1d, c1d)
  cnt_pair = _cntpair_call(_countacc_call(dst))
  s1_pair = _conv_call(t1_pair.reshape(N, HID), src, dst).reshape(NP, PW)
  a1_pair, stats = _meanstats_call(s1_pair, t1_pair, cnt_pair)
  t2_pair = _bnmat_call(a1_pair, stats, w2d)
  s2_pair = _conv_call(t2_pair.reshape(N, HID), src, dst).reshape(NP, PW)
  out = _pool_call(s2_pair, t2_pair, cnt_pair, batch_e, batch_o, b2[None, :],
                   Wc1.T, bc1[None, :], Wc2.T, bc2[None, :])
  return out


# speedup vs baseline: 21.7556x; 1.0585x over previous
import jax
import jax.numpy as jnp
from jax import lax
from jax.experimental import pallas as pl
from jax.experimental.pallas import tpu as pltpu
from jax.experimental.pallas import tpu_sc as plsc

N = 50000
E = 800000
IN_DIM = 128
HID = 64
OUT_DIM = 2
G = 64
EPS = 1e-5

NC = 2
NT = 16
NPARTS = 2
HP = HID // NPARTS
NP = N // 2
PW = 2 * HID

ROWS_MAIN = 3136
ROWS_TAIL = N - (NT - 1) * ROWS_MAIN

KE = 2000
EDGES_PER_TILE = E // NT
CONV_CHUNKS = EDGES_PER_TILE // KE

KC = 1000
EDGES_PER_WORKER = E // (NC * NT)
CNT_CHUNKS = EDGES_PER_WORKER // KC
CW = 16

_MESH = dict(core_axis_name="c", subcore_axis_name="s")


def _zero_vmem_rows(ref, nrows, width):
  zv = jnp.zeros((16,), jnp.float32)

  def body(i, _):
    for off in range(0, width, 16):
      ref[i, pl.ds(off, 16)] = zv
    return 0

  lax.fori_loop(0, nrows, body, 0)


def _fill_vmem_rows(ref, nrows, width, value):
  vv = jnp.full((16,), value, jnp.float32)

  def body(i, _):
    for off in range(0, width, 16):
      ref[i, pl.ds(off, 16)] = vv
    return 0

  lax.fori_loop(0, nrows, body, 0)


def _countacc_body(dst_ref, out_ref, idx_v0, idx_v1, ones_v, zrows_v, sem_i0,
                   sem_i1, sem_s0, sem_s1, accum):
  cid = lax.axis_index("c")
  sid = lax.axis_index("s")
  _zero_vmem_rows(zrows_v, ROWS_MAIN, CW)
  _fill_vmem_rows(ones_v, KC, CW, 1.0)
  row0 = sid * ROWS_MAIN
  idx_v = (idx_v0, idx_v1)
  sem_i = (sem_i0, sem_i1)
  sem_s = (sem_s0, sem_s1)

  @pl.when(sid < NT - 1)
  def _():
    pltpu.sync_copy(zrows_v, accum.at[pl.ds(row0, ROWS_MAIN)])

  @pl.when(sid == NT - 1)
  def _():
    pltpu.sync_copy(
        zrows_v.at[pl.ds(0, ROWS_TAIL)], accum.at[pl.ds(row0, ROWS_TAIL)]
    )

  plsc.subcore_barrier()

  wid = cid * NT + sid

  def idx_start(j, b):
    base = wid * EDGES_PER_WORKER + j * KC
    pltpu.make_async_copy(
        dst_ref.at[pl.ds(base, KC)], idx_v[b], sem_i[b]
    ).start()

  def idx_wait(j, b):
    base = wid * EDGES_PER_WORKER + j * KC
    pltpu.make_async_copy(
        dst_ref.at[pl.ds(base, KC)], idx_v[b], sem_i[b]
    ).wait()

  idx_start(0, 0)

  def step(j, _):
    for b in range(2):

      @pl.when(j % 2 == b)
      def _():
        nb = 1 - b
        idx_wait(j, b)

        @pl.when(j >= 1)
        def _():
          pltpu.make_async_copy(
              ones_v, accum.at[idx_v[nb]], sem_s[nb]
          ).wait()

        @pl.when(j < CNT_CHUNKS - 1)
        def _():
          idx_start(j + 1, nb)

        pltpu.make_async_copy(
            ones_v, accum.at[idx_v[b]], sem_s[b]
        ).start(add=True)

    return 0

  lax.fori_loop(0, CNT_CHUNKS, step, 0)
  lastb = (CNT_CHUNKS - 1) % 2
  pltpu.make_async_copy(
      ones_v, accum.at[idx_v[lastb]], sem_s[lastb]
  ).wait()
  plsc.subcore_barrier()

  @pl.when(sid < NT - 1)
  def _():
    pltpu.sync_copy(
        accum.at[pl.ds(row0, ROWS_MAIN)],
        out_ref.at[cid].at[pl.ds(row0, ROWS_MAIN)],
    )

  @pl.when(sid == NT - 1)
  def _():
    pltpu.sync_copy(
        accum.at[pl.ds(row0, ROWS_TAIL)],
        out_ref.at[cid].at[pl.ds(row0, ROWS_TAIL)],
    )


def _countacc_call(dst):
  kern = pl.kernel(
      _countacc_body,
      out_type=jax.ShapeDtypeStruct((NC, N, CW), jnp.float32),
      mesh=plsc.VectorSubcoreMesh(**_MESH),
      compiler_params=pltpu.CompilerParams(use_tc_tiling_on_sc=False),
      scratch_types=[
          pltpu.VMEM((KC,), jnp.int32),
          pltpu.VMEM((KC,), jnp.int32),
          pltpu.VMEM((KC, CW), jnp.float32),
          pltpu.VMEM((ROWS_MAIN, CW), jnp.float32),
          pltpu.SemaphoreType.DMA,
          pltpu.SemaphoreType.DMA,
          pltpu.SemaphoreType.DMA,
          pltpu.SemaphoreType.DMA,
          pltpu.VMEM_SHARED((N, CW), jnp.float32),
      ],
  )
  return kern(dst)


def _cntpair_body(cp_ref, out_ref, z0_v, z1_v, pair_v):
  cid = lax.axis_index("c")
  sid = lax.axis_index("s")
  wid = cid * NT + sid
  zone = 784
  half = 392
  pr0 = wid * zone
  tail_rem = NP - 31 * zone - half

  def emit(local_off, nrows):
    n0 = 2 * (pr0 + local_off)
    pltpu.sync_copy(cp_ref.at[0].at[pl.ds(n0, 2 * nrows)],
                    z0_v.at[pl.ds(0, 2 * nrows)])
    pltpu.sync_copy(cp_ref.at[1].at[pl.ds(n0, 2 * nrows)],
                    z1_v.at[pl.ds(0, 2 * nrows)])

    def fill(i, _):
      v0 = z0_v[2 * i, pl.ds(0, CW)] + z1_v[2 * i, pl.ds(0, CW)] + 1.0
      v1 = (
          z0_v[2 * i + 1, pl.ds(0, CW)] + z1_v[2 * i + 1, pl.ds(0, CW)] + 1.0
      )
      for u in range(4):
        pair_v[i, pl.ds(u * CW, CW)] = v0
      for u in range(4, 8):
        pair_v[i, pl.ds(u * CW, CW)] = v1
      return 0

    lax.fori_loop(0, nrows, fill, 0)
    pltpu.sync_copy(
        pair_v.at[pl.ds(0, nrows)], out_ref.at[pl.ds(pr0 + local_off, nrows)]
    )

  emit(0, half)

  @pl.when(wid < NC * NT - 1)
  def _():
    emit(half, half)

  @pl.when(wid == NC * NT - 1)
  def _():
    emit(half, tail_rem)


def _cntpair_call(cparts):
  kern = pl.kernel(
      _cntpair_body,
      out_type=jax.ShapeDtypeStruct((NP, PW), jnp.float32),
      mesh=plsc.VectorSubcoreMesh(**_MESH),
      compiler_params=pltpu.CompilerParams(use_tc_tiling_on_sc=False),
      scratch_types=[
          pltpu.VMEM((2 * 392, CW), jnp.float32),
          pltpu.VMEM((2 * 392, CW), jnp.float32),
          pltpu.VMEM((392, PW), jnp.float32),
      ],
  )
  return kern(cparts)


RC = 1000


def _reformat_body(table_ref, tpart_ref, part_v0, part_v1, sem0, sem1):
  cid = lax.axis_index("c")
  sid = lax.axis_index("s")
  wid = cid * NT + sid
  zone = 1568
  row0 = wid * zone
  tail = N - 31 * zone
  part_v = (part_v0, part_v1)
  sems = (sem0, sem1)

  def emit(q, nrows):
    b = q % 2
    pltpu.make_async_copy(
        table_ref.at[pl.ds(row0, nrows), pl.ds(q * HP, HP)],
        part_v[b].at[pl.ds(0, nrows)],
        sems[b],
    ).start()

  def drain(q, nrows):
    b = q % 2
    pltpu.make_async_copy(
        table_ref.at[pl.ds(row0, nrows), pl.ds(q * HP, HP)],
        part_v[b].at[pl.ds(0, nrows)],
        sems[b],
    ).wait()
    pltpu.sync_copy(
        part_v[b].at[pl.ds(0, nrows)],
        tpart_ref.at[q].at[pl.ds(row0, nrows)],
    )

  def go(nrows):
    emit(0, nrows)
    emit(1, nrows)
    for q in range(NPARTS):
      drain(q, nrows)
      if q + 2 < NPARTS:
        emit(q + 2, nrows)

  @pl.when(wid < NC * NT - 1)
  def _():
    go(zone)

  @pl.when(wid == NC * NT - 1)
  def _():
    go(tail)


def _reformat_call(table):
  kern = pl.kernel(
      _reformat_body,
      out_type=jax.ShapeDtypeStruct((NPARTS, N, HP), jnp.bfloat16),
      mesh=plsc.VectorSubcoreMesh(**_MESH),
      compiler_params=pltpu.CompilerParams(use_tc_tiling_on_sc=False),
      scratch_types=[
          pltpu.VMEM((1568, HP), jnp.bfloat16),
          pltpu.VMEM((1568, HP), jnp.bfloat16),
          pltpu.SemaphoreType.DMA,
          pltpu.SemaphoreType.DMA,
      ],
  )
  return kern(table)


def _conv_body(tpart_ref, src_ref, dst_ref, out_ref, src_v0, src_v1, src_v2,
               dst_v0, dst_v1, dst_v2, rows_v0, rows_v1, sem_i0, sem_i1,
               sem_i2, sem_g0, sem_g1, sem_s0, sem_s1, accum):
  cid = lax.axis_index("c")
  sid = lax.axis_index("s")
  row0 = sid * ROWS_MAIN
  src_v = (src_v0, src_v1, src_v2)
  dst_v = (dst_v0, dst_v1, dst_v2)
  rows_v = (rows_v0, rows_v1)
  sem_i = (sem_i0, sem_i1, sem_i2)
  sem_g = (sem_g0, sem_g1)
  sem_s = (sem_s0, sem_s1)

  def idx_start(j, b3):
    base = sid * EDGES_PER_TILE + j * KE
    pltpu.make_async_copy(
        src_ref.at[pl.ds(base, KE)], src_v[b3], sem_i[b3]
    ).start()
    pltpu.make_async_copy(
        dst_ref.at[pl.ds(base, KE)], dst_v[b3], sem_i[b3]
    ).start()

  def idx_wait(j, b3):
    base = sid * EDGES_PER_TILE + j * KE
    pltpu.make_async_copy(
        src_ref.at[pl.ds(base, KE)], src_v[b3], sem_i[b3]
    ).wait()
    pltpu.make_async_copy(
        dst_ref.at[pl.ds(base, KE)], dst_v[b3], sem_i[b3]
    ).wait()

  part = cid
  col0 = part * HP
  zb = jnp.zeros((32,), jnp.bfloat16)

  def zrow(i, _):
    rows_v0[i, pl.ds(0, HP)] = zb
    return 0

  lax.fori_loop(0, KE, zrow, 0)
  pltpu.sync_copy(rows_v0, accum.at[pl.ds(row0, KE)])

  @pl.when(sid < NT - 1)
  def _():
    pltpu.sync_copy(
        rows_v0.at[pl.ds(0, ROWS_MAIN - KE)],
        accum.at[pl.ds(row0 + KE, ROWS_MAIN - KE)],
    )

  @pl.when(sid == NT - 1)
  def _():
    pltpu.sync_copy(
        rows_v0.at[pl.ds(0, ROWS_TAIL - KE)],
        accum.at[pl.ds(row0 + KE, ROWS_TAIL - KE)],
    )

  plsc.subcore_barrier()

  def gather_start(s3, b):
    pltpu.make_async_copy(
        tpart_ref.at[part].at[src_v[s3]], rows_v[b], sem_g[b]
    ).start()

  def gather_wait(s3, b):
    pltpu.make_async_copy(
        tpart_ref.at[part].at[src_v[s3]], rows_v[b], sem_g[b]
    ).wait()

  def scatter_start(s3, b):
    pltpu.make_async_copy(
        rows_v[b], accum.at[dst_v[s3]], sem_s[b]
    ).start(add=True)

  def scatter_wait(s3, b):
    pltpu.make_async_copy(
        rows_v[b], accum.at[dst_v[s3]], sem_s[b]
    ).wait()

  idx_start(0, 0)
  idx_wait(0, 0)
  gather_start(0, 0)
  idx_start(1, 1)

  def step(j, _):
    for m in range(6):

      @pl.when(j % 6 == m)
      def _():
        b = m % 2
        nb = 1 - b
        s_j = m % 3
        s_j1 = (m + 1) % 3
        s_j2 = (m + 2) % 3
        gather_wait(s_j, b)
        scatter_start(s_j, b)

        @pl.when(j + 1 < CONV_CHUNKS)
        def _():
          idx_wait(j + 1, s_j1)

          @pl.when(j >= 1)
          def _():
            scatter_wait(s_j2, nb)

          gather_start(s_j1, nb)

        @pl.when(j + 2 < CONV_CHUNKS)
        def _():
          idx_start(j + 2, s_j2)

    return 0

  lax.fori_loop(0, CONV_CHUNKS, step, 0)
  scatter_wait((CONV_CHUNKS - 2) % 3, (CONV_CHUNKS - 2) % 2)
  scatter_wait((CONV_CHUNKS - 1) % 3, (CONV_CHUNKS - 1) % 2)
  plsc.subcore_barrier()

  @pl.when(sid < NT - 1)
  def _():
    pltpu.sync_copy(
        accum.at[pl.ds(row0, ROWS_MAIN)],
        out_ref.at[pl.ds(row0, ROWS_MAIN), pl.ds(col0, HP)],
    )

  @pl.when(sid == NT - 1)
  def _():
    pltpu.sync_copy(
        accum.at[pl.ds(row0, ROWS_TAIL)],
        out_ref.at[pl.ds(row0, ROWS_TAIL), pl.ds(col0, HP)],
    )


def _conv_call(table, src, dst):
  tpart = _reformat_call(table)
  kern = pl.kernel(
      _conv_body,
      out_type=jax.ShapeDtypeStruct((N, HID), jnp.bfloat16),
      mesh=plsc.VectorSubcoreMesh(**_MESH),
      compiler_params=pltpu.CompilerParams(use_tc_tiling_on_sc=False),
      scratch_types=[
          pltpu.VMEM((KE,), jnp.int32),
          pltpu.VMEM((KE,), jnp.int32),
          pltpu.VMEM((KE,), jnp.int32),
          pltpu.VMEM((KE,), jnp.int32),
          pltpu.VMEM((KE,), jnp.int32),
          pltpu.VMEM((KE,), jnp.int32),
          pltpu.VMEM((KE, HP), jnp.bfloat16),
          pltpu.VMEM((KE, HP), jnp.bfloat16),
          pltpu.SemaphoreType.DMA,
          pltpu.SemaphoreType.DMA,
          pltpu.SemaphoreType.DMA,
          pltpu.SemaphoreType.DMA,
          pltpu.SemaphoreType.DMA,
          pltpu.SemaphoreType.DMA,
          pltpu.SemaphoreType.DMA,
          pltpu.VMEM_SHARED((N, HP), jnp.bfloat16),
      ],
  )
  return kern(tpart, src, dst)


BP = 1000
NBLK = NP // BP


def _encoder_kernel(x_ref, wenc_ref, benc_ref, a1_ref, c1_ref, out_ref):
  r = jnp.maximum(
      jnp.dot(x_ref[...], wenc_ref[...], preferred_element_type=jnp.float32)
      + benc_ref[...],
      0.0,
  )
  out_ref[...] = (
      jnp.dot(r, a1_ref[...], preferred_element_type=jnp.float32) + c1_ref[...]
  ).astype(jnp.bfloat16)


def _encoder_call(x_pair, wenc2, benc2, a1d, c1d):
  return pl.pallas_call(
      _encoder_kernel,
      grid=(NBLK,),
      in_specs=[
          pl.BlockSpec((BP, 2 * IN_DIM), lambda i: (i, 0)),
          pl.BlockSpec((2 * IN_DIM, PW), lambda i: (0, 0)),
          pl.BlockSpec((1, PW), lambda i: (0, 0)),
          pl.BlockSpec((PW, PW), lambda i: (0, 0)),
          pl.BlockSpec((1, PW), lambda i: (0, 0)),
      ],
      out_specs=pl.BlockSpec((BP, PW), lambda i: (i, 0)),
      out_shape=jax.ShapeDtypeStruct((NP, PW), jnp.bfloat16),
  )(x_pair, wenc2, benc2, a1d, c1d)


def _meanstats_kernel(s_ref, t_ref, cnt_ref, a_ref, stats_ref):
  i = pl.program_id(0)
  m = (
      s_ref[...].astype(jnp.float32) + t_ref[...].astype(jnp.float32)
  ) / cnt_ref[...]
  a_ref[...] = m.astype(jnp.bfloat16)
  part = jnp.concatenate(
      [
          jnp.sum(m, axis=0, keepdims=True),
          jnp.sum(m * m, axis=0, keepdims=True),
      ],
      axis=0,
  )

  @pl.when(i == 0)
  def _():
    stats_ref[...] = part

  @pl.when(i > 0)
  def _():
    stats_ref[...] += part


def _meanstats_call(s_pair, t_pair, cnt_pair):
  return pl.pallas_call(
      _meanstats_kernel,
      grid=(NBLK,),
      in_specs=[
          pl.BlockSpec((BP, PW), lambda i: (i, 0)),
          pl.BlockSpec((BP, PW), lambda i: (i, 0)),
          pl.BlockSpec((BP, PW), lambda i: (i, 0)),
      ],
      out_specs=[
          pl.BlockSpec((BP, PW), lambda i: (i, 0)),
          pl.BlockSpec((2, PW), lambda i: (0, 0)),
      ],
      out_shape=[
          jax.ShapeDtypeStruct((NP, PW), jnp.bfloat16),
          jax.ShapeDtypeStruct((2, PW), jnp.float32),
      ],
  )(s_pair, t_pair, cnt_pair)


def _bnmat_kernel(a_ref, stats_ref, w2_ref, out_ref):
  mean = (stats_ref[0:1, 0:HID] + stats_ref[0:1, HID:PW]) / N
  msq = (stats_ref[1:2, 0:HID] + stats_ref[1:2, HID:PW]) / N
  var = jnp.maximum(msq - mean * mean, 0.0)
  scale = lax.rsqrt(var + EPS)
  mean2 = jnp.concatenate([mean, mean], axis=1)
  scale2 = jnp.concatenate([scale, scale], axis=1)
  h = jnp.maximum((a_ref[...].astype(jnp.float32) - mean2) * scale2, 0.0)
  out_ref[...] = jnp.dot(
      h, w2_ref[...], preferred_element_type=jnp.float32
  ).astype(jnp.bfloat16)


def _bnmat_call(a_pair, stats, w2d):
  return pl.pallas_call(
      _bnmat_kernel,
      grid=(NBLK,),
      in_specs=[
          pl.BlockSpec((BP, PW), lambda i: (i, 0)),
          pl.BlockSpec((2, PW), lambda i: (0, 0)),
          pl.BlockSpec((PW, PW), lambda i: (0, 0)),
      ],
      out_specs=pl.BlockSpec((BP, PW), lambda i: (i, 0)),
      out_shape=jax.ShapeDtypeStruct((NP, PW), jnp.bfloat16),
  )(a_pair, stats, w2d)


def _pool_kernel(s_ref, t_ref, cnt_ref, be_ref, bo_ref, b2_ref, wc1_ref,
                 bc1_ref, wc2_ref, bc2_ref, out_ref, acc_ref):
  i = pl.program_id(0)
  h = (
      s_ref[...].astype(jnp.float32) + t_ref[...].astype(jnp.float32)
  ) / cnt_ref[...]
  be = jnp.reshape(be_ref[0], (1, BP))
  bo = jnp.reshape(bo_ref[0], (1, BP))
  giota = lax.broadcasted_iota(jnp.int32, (G, BP), 0)
  ohe = (giota == be).astype(jnp.float32)
  oho = (giota == bo).astype(jnp.float32)
  ones = jnp.ones((BP, HID), jnp.float32)
  he = jnp.concatenate([h[:, 0:HID], ones], axis=1)
  ho = jnp.concatenate([h[:, HID:PW], ones], axis=1)
  part = (
      jnp.dot(ohe, he, preferred_element_type=jnp.float32)
      + jnp.dot(oho, ho, preferred_element_type=jnp.float32)
  )

  @pl.when(i == 0)
  def _():
    acc_ref[...] = part

  @pl.when(i > 0)
  def _():
    acc_ref[...] += part

  @pl.when(i == NBLK - 1)
  def _():
    sums = acc_ref[:, 0:HID]
    gcnt = acc_ref[:, HID:HID + 1]
    pm = sums / jnp.maximum(gcnt, 1.0)
    pm = pm + jnp.where(gcnt > 0.0, 1.0, 0.0) * b2_ref[...]
    z = jnp.maximum(
        jnp.dot(pm, wc1_ref[...], preferred_element_type=jnp.float32)
        + bc1_ref[...],
        0.0,
    )
    out_ref[...] = (
        jnp.dot(z, wc2_ref[...], preferred_element_type=jnp.float32)
        + bc2_ref[...]
    )


def _pool_call(s_pair, t_pair, cnt_pair, batch_e, batch_o, b2, Wc1T, bc1,
               Wc2T, bc2):
  return pl.pallas_call(
      _pool_kernel,
      grid=(NBLK,),
      in_specs=[
          pl.BlockSpec((BP, PW), lambda i: (i, 0)),
          pl.BlockSpec((BP, PW), lambda i: (i, 0)),
          pl.BlockSpec((BP, PW), lambda i: (i, 0)),
          pl.BlockSpec((1, 1, BP), lambda i: (i, 0, 0)),
          pl.BlockSpec((1, 1, BP), lambda i: (i, 0, 0)),
          pl.BlockSpec((1, HID), lambda i: (0, 0)),
          pl.BlockSpec((HID, HID), lambda i: (0, 0)),
          pl.BlockSpec((1, HID), lambda i: (0, 0)),
          pl.BlockSpec((HID, OUT_DIM), lambda i: (0, 0)),
          pl.BlockSpec((1, OUT_DIM), lambda i: (0, 0)),
      ],
      out_specs=pl.BlockSpec((G, OUT_DIM), lambda i: (0, 0)),
      out_shape=jax.ShapeDtypeStruct((G, OUT_DIM), jnp.float32),
      scratch_shapes=[pltpu.VMEM((G, 2 * HID), jnp.float32)],
  )(s_pair, t_pair, cnt_pair, batch_e, batch_o, b2, Wc1T, bc1, Wc2T, bc2)


def _blockdiag(w):
  z = jnp.zeros_like(w)
  return jnp.concatenate(
      [jnp.concatenate([w, z], axis=1), jnp.concatenate([z, w], axis=1)],
      axis=0,
  )


def kernel(x, edge_index, batch, W_enc, b_enc, bn_gamma, bn_beta,
           W1, b1, W2, b2, Wc1, bc1, Wc2, bc2):
  g = bn_gamma / jnp.sqrt(1.0 + EPS)
  A1 = g[:, None] * W1.T
  c1 = bn_beta @ W1.T
  src = edge_index[0]
  dst = edge_index[1]

  x_pair = x.reshape(NP, 2 * IN_DIM)
  wenc2 = _blockdiag(W_enc.T)
  benc2 = jnp.tile(b_enc, 2)[None, :]
  a1d = _blockdiag(A1)
  c1d = jnp.tile(c1, 2)[None, :]
  w2d = _blockdiag(W2.T)
  batch_e = batch[0::2].reshape(NBLK, 1, BP)
  batch_o = batch[1::2].reshape(NBLK, 1, BP)

  t1_pair = _encoder_call(x_pair, wenc2, benc2, a1d, c1d)
  cnt_pair = _cntpair_call(_countacc_call(dst))
  s1_pair = _conv_call(t1_pair.reshape(N, HID), src, dst).reshape(NP, PW)
  a1_pair, stats = _meanstats_call(s1_pair, t1_pair, cnt_pair)
  t2_pair = _bnmat_call(a1_pair, stats, w2d)
  s2_pair = _conv_call(t2_pair.reshape(N, HID), src, dst).reshape(NP, PW)
  out = _pool_call(s2_pair, t2_pair, cnt_pair, batch_e, batch_o, b2[None, :],
                   Wc1.T, bc1[None, :], Wc2.T, bc2[None, :])
  return out
